# Initial kernel scaffold; baseline (speedup 1.0000x reference)
import jax
import jax.numpy as jnp
from jax.experimental import pallas as pl


def kernel(x, edge_list, batch, W1, b1, W2, b2, W3, b3, W4, b4, W5, b5, pw1, pb1, pw2, pb2, fw1, fb1, fw2, fb2, y):
    raise NotImplementedError("write your pallas kernel here")


# speedup vs baseline: 1.0636x; 1.0636x over previous
import jax
import jax.numpy as jnp
from jax.experimental import pallas as pl

N = 10000
E = 160000
NEG = -1e30


def _mm_body(x_ref, w_ref, o_ref):
    o_ref[...] = jnp.dot(x_ref[...], w_ref[...],
                         preferred_element_type=jnp.float32)


def _mm(x, w, bm=512):
    m, k = x.shape
    nc = w.shape[1]
    mp = ((m + bm - 1) // bm) * bm
    ncp = ((nc + 127) // 128) * 128
    xp = jnp.pad(x, ((0, mp - m), (0, 0))) if mp != m else x
    wp = jnp.pad(w, ((0, 0), (0, ncp - nc))) if ncp != nc else w
    out = pl.pallas_call(
        _mm_body,
        grid=(mp // bm,),
        in_specs=[pl.BlockSpec((bm, k), lambda i: (i, 0)),
                  pl.BlockSpec((k, ncp), lambda i: (0, 0))],
        out_specs=pl.BlockSpec((bm, ncp), lambda i: (i, 0)),
        out_shape=jax.ShapeDtypeStruct((mp, ncp), jnp.float32),
    )(xp, wp)
    return out[:m, :nc]


def _gcn_conv(x, ei, W, b):
    n = x.shape[0]
    loop = jnp.arange(n)
    row = jnp.concatenate([ei[0], loop])
    col = jnp.concatenate([ei[1], loop])
    deg = jnp.zeros((n,), x.dtype).at[col].add(1.0)
    dis = jax.lax.rsqrt(deg)
    norm = dis[row] * dis[col]
    xw = _mm(x, W)
    out = jnp.zeros((n, W.shape[1]), x.dtype).at[col].add(norm[:, None] * xw[row])
    return out + b


def _edge_scores(x, ei, pw, pb):
    h = x.shape[1]
    a = _mm(x, pw[:h]).reshape(-1)
    bvec = _mm(x, pw[h:]).reshape(-1)
    return jax.nn.sigmoid(a[ei[0]] + bvec[ei[1]] + pb[0])


def _cluster_structure(s, ei, n):
    mask = s > 0.5
    src, dst = ei[0], ei[1]
    labels = jnp.arange(n)

    def step(lbl):
        new = lbl.at[dst].min(jnp.where(mask, lbl[src], n))
        new = new.at[src].min(jnp.where(mask, new[dst], n))
        return jnp.minimum(new, lbl)

    def cond(carry):
        lbl, new = carry
        return jnp.any(new != lbl)

    def body(carry):
        lbl, new = carry
        return new, step(new)

    lbl, new = jax.lax.while_loop(cond, body, (labels, step(labels)))
    return mask, new, n


def _cluster_pool_apply(x, ei, batch, s, mask, inv, nc):
    n = x.shape[0]
    sm = jnp.where(mask, s, NEG)
    ns = jnp.full((n,), NEG, x.dtype).at[ei[0]].max(sm).at[ei[1]].max(sm)
    ns = jnp.where(ns <= NEG * 0.5, 1.0, ns)
    x_new = jnp.zeros((nc, x.shape[1]), x.dtype).at[inv].add(x * ns[:, None])
    ei_new = inv[ei]
    batch_new = jnp.zeros((nc,), batch.dtype).at[inv].max(batch)
    return x_new, ei_new, batch_new


def kernel(x, edge_list, batch, W1, b1, W2, b2, W3, b3, W4, b4, W5, b5,
           pw1, pb1, pw2, pb2, fw1, fb1, fw2, fb2, y=None):
    ei = edge_list.T
    h = jax.nn.relu(_gcn_conv(x, ei, W1, b1))
    h = jax.nn.relu(_gcn_conv(h, ei, W2, b2))
    s1 = _edge_scores(h, ei, pw1, pb1)
    st1 = _cluster_structure(s1, ei, h.shape[0])
    valid = jnp.zeros((st1[2],), jnp.int32).at[st1[1]].set(1)
    h, ei2, bt2 = _cluster_pool_apply(h, ei, batch, s1, *st1)
    h = jax.nn.relu(_gcn_conv(h, ei2, W3, b3))
    h = jax.nn.relu(_gcn_conv(h, ei2, W4, b4))
    s2 = _edge_scores(h, ei2, pw2, pb2)
    st2 = _cluster_structure(s2, ei2, h.shape[0])
    valid = jnp.zeros((st2[2],), jnp.int32).at[st2[1]].max(valid)
    h, ei3, bt3 = _cluster_pool_apply(h, ei2, bt2, s2, *st2)
    h = jax.nn.relu(_gcn_conv(h, ei3, W5, b5))
    vf = valid.astype(h.dtype)
    gsum = jnp.zeros((1, h.shape[1]), h.dtype).at[bt3].add(h * vf[:, None])
    cnt = jnp.zeros((1,), h.dtype).at[bt3].add(vf)
    g = gsum / jnp.maximum(cnt, 1.0)[:, None]
    g = jax.nn.relu(_mm(g, fw1, bm=8) + fb1)
    out = jax.nn.sigmoid(_mm(g, fw2, bm=8) + fb2)
    return out.reshape(-1)


# speedup vs baseline: 1.2382x; 1.1642x over previous
import functools

import jax
import jax.numpy as jnp
from jax import lax
from jax.experimental import pallas as pl
from jax.experimental.pallas import tpu as pltpu
from jax.experimental.pallas import tpu_sc as plsc

N = 10000
NP = 10240
E = 160000
D = 128
NEG = -1e30
_STRIPE = NP // 16


def _mm_body(x_ref, w_ref, o_ref):
    o_ref[...] = jnp.dot(x_ref[...], w_ref[...],
                         preferred_element_type=jnp.float32)


def _mm(x, w, bm=512):
    m, k = x.shape
    nc = w.shape[1]
    mp = ((m + bm - 1) // bm) * bm
    ncp = ((nc + 127) // 128) * 128
    xp = jnp.pad(x, ((0, mp - m), (0, 0))) if mp != m else x
    wp = jnp.pad(w, ((0, 0), (0, ncp - nc))) if ncp != nc else w
    out = pl.pallas_call(
        _mm_body,
        grid=(mp // bm,),
        in_specs=[pl.BlockSpec((bm, k), lambda i: (i, 0)),
                  pl.BlockSpec((k, ncp), lambda i: (0, 0))],
        out_specs=pl.BlockSpec((bm, ncp), lambda i: (i, 0)),
        out_shape=jax.ShapeDtypeStruct((mp, ncp), jnp.float32),
    )(xp, wp)
    return out[:m, :nc]


def _mm_conv_body(x_ref, w_ref, dis_ref, xw_ref, xws_ref):
    t = jnp.dot(x_ref[...], w_ref[...], preferred_element_type=jnp.float32)
    xw_ref[...] = t
    xws_ref[...] = t * dis_ref[...]


def _mm_conv(x, w, dis_col, bm=1024):
    k = x.shape[1]
    outs = pl.pallas_call(
        _mm_conv_body,
        grid=(NP // bm,),
        in_specs=[pl.BlockSpec((bm, k), lambda i: (i, 0)),
                  pl.BlockSpec((k, D), lambda i: (0, 0)),
                  pl.BlockSpec((bm, 1), lambda i: (i, 0))],
        out_specs=[pl.BlockSpec((bm, D), lambda i: (i, 0)),
                   pl.BlockSpec((bm, D), lambda i: (i, 0))],
        out_shape=[jax.ShapeDtypeStruct((NP, D), jnp.float32),
                   jax.ShapeDtypeStruct((NP, D), jnp.float32)],
    )(x, w, dis_col)
    return outs


def _elt_post_body(a_ref, b_ref, xw_ref, dis_ref, iv_ref, bias_ref, o_ref):
    agg = (a_ref[...] + b_ref[...]) * dis_ref[...]
    o_ref[...] = jnp.maximum(agg + xw_ref[...] * iv_ref[...] + bias_ref[...],
                             0.0)


def _elt_post(acc2, xw, dis_col, iv_col, bias, bm=1024):
    return pl.pallas_call(
        _elt_post_body,
        grid=(NP // bm,),
        in_specs=[pl.BlockSpec((bm, D), lambda i: (i, 0)),
                  pl.BlockSpec((bm, D), lambda i: (i, 0)),
                  pl.BlockSpec((bm, D), lambda i: (i, 0)),
                  pl.BlockSpec((bm, 1), lambda i: (i, 0)),
                  pl.BlockSpec((bm, 1), lambda i: (i, 0)),
                  pl.BlockSpec((1, D), lambda i: (0, 0))],
        out_specs=pl.BlockSpec((bm, D), lambda i: (i, 0)),
        out_shape=jax.ShapeDtypeStruct((NP, D), jnp.float32),
    )(acc2[0], acc2[1], xw, dis_col, iv_col, bias.reshape(1, D))


def _spmm_body(nchunks, xs_ref, src_ref, dst_ref, zrow_ref, out_ref,
               idx_s, idx_d, rows, acc, sem, semz):
    cid = lax.axis_index("c")
    sid = lax.axis_index("s")
    stripe = sid * _STRIPE

    for i in range(_STRIPE // 128):
        pltpu.async_copy(zrow_ref, acc.at[pl.ds(stripe + i * 128, 128)],
                         semz).wait()
    plsc.subcore_barrier()

    half = nchunks // 2
    cnt = (half - sid + 15) // 16

    def body(j, carry):
        c = 2 * (16 * j + sid) + cid
        pltpu.sync_copy(src_ref.at[c], idx_s)
        pltpu.async_copy(xs_ref.at[idx_s], rows, sem).wait()
        pltpu.sync_copy(dst_ref.at[c], idx_d)
        pltpu.sync_copy(rows, acc.at[idx_d], add=True)
        return carry

    lax.fori_loop(0, cnt, body, 0)
    plsc.subcore_barrier()

    pltpu.sync_copy(acc.at[pl.ds(stripe, _STRIPE)],
                    out_ref.at[pl.ds(cid * NP + stripe, _STRIPE)])


def _sc_spmm(xs, src_c, dst_c, zrow, nchunks):
    mesh = plsc.VectorSubcoreMesh(core_axis_name="c", subcore_axis_name="s")
    flat = pl.kernel(
        functools.partial(_spmm_body, nchunks),
        out_type=jax.ShapeDtypeStruct((2 * NP, D), jnp.float32),
        mesh=mesh,
        scratch_types=[
            pltpu.VMEM((128,), jnp.int32),
            pltpu.VMEM((128,), jnp.int32),
            pltpu.VMEM((128, D), jnp.float32),
            pltpu.VMEM_SHARED((NP, D), jnp.float32),
            pltpu.SemaphoreType.DMA,
            pltpu.SemaphoreType.DMA,
        ],
    )(xs, src_c, dst_c, zrow)
    return flat.reshape(2, NP, D)


def _gcn_conv_sc(h, src_c, dst_c, zrow, dis_col, iv_col, W, b):
    xw, xws = _mm_conv(h, W, dis_col)
    acc2 = _sc_spmm(xws, src_c, dst_c, zrow, src_c.shape[0])
    return _elt_post(acc2, xw, dis_col, iv_col, b)


def _edge_scores(x, ei, pw, pb):
    h = x.shape[1]
    a = _mm(x, pw[:h]).reshape(-1)
    bvec = _mm(x, pw[h:]).reshape(-1)
    return jax.nn.sigmoid(a[ei[0]] + bvec[ei[1]] + pb[0])


def _cluster_structure(s, ei, n):
    mask = s > 0.5
    src, dst = ei[0], ei[1]
    labels = jnp.arange(n)

    def step(lbl):
        new = lbl.at[dst].min(jnp.where(mask, lbl[src], n))
        new = new.at[src].min(jnp.where(mask, new[dst], n))
        return jnp.minimum(new, lbl)

    def cond(carry):
        lbl, new = carry
        return jnp.any(new != lbl)

    def body(carry):
        lbl, new = carry
        return new, step(new)

    lbl, new = jax.lax.while_loop(cond, body, (labels, step(labels)))
    return mask, new


def kernel(x, edge_list, batch, W1, b1, W2, b2, W3, b3, W4, b4, W5, b5,
           pw1, pb1, pw2, pb2, fw1, fb1, fw2, fb2, y=None):
    src = edge_list[:, 0].astype(jnp.int32)
    dst = edge_list[:, 1].astype(jnp.int32)
    src_c = src.reshape(E // 128, 128)
    dst_c = dst.reshape(E // 128, 128)
    zrow = jnp.zeros((128, D), jnp.float32)
    x_p = jnp.pad(x, ((0, NP - N), (0, 0)))

    def level_norm(dstv):
        deg = jnp.zeros((N,), jnp.float32).at[dstv].add(1.0) + 1.0
        deg_p = jnp.pad(deg, (0, NP - N), constant_values=1.0)
        return jax.lax.rsqrt(deg_p)[:, None], (1.0 / deg_p)[:, None]

    dis1, iv1 = level_norm(dst)
    h = _gcn_conv_sc(x_p, src_c, dst_c, zrow, dis1, iv1, W1, b1)
    h = _gcn_conv_sc(h, src_c, dst_c, zrow, dis1, iv1, W2, b2)

    hr = h[:N]
    ei = jnp.stack([src, dst])
    s1 = _edge_scores(hr, ei, pw1, pb1)
    mask1, inv1 = _cluster_structure(s1, ei, N)
    valid = jnp.zeros((N,), jnp.int32).at[inv1].set(1)

    sm = jnp.where(mask1, s1, NEG)
    ns = jnp.full((N,), NEG, jnp.float32).at[src].max(sm).at[dst].max(sm)
    ns = jnp.where(ns <= NEG * 0.5, 1.0, ns)
    hs = hr * ns[:, None]
    hs_p = jnp.pad(hs, ((0, NP - N), (0, 0)))
    inv_p = jnp.concatenate([inv1, jnp.arange(N, NP, dtype=jnp.int32)])
    pool_src = jnp.arange(NP, dtype=jnp.int32).reshape(NP // 128, 128)
    pool_dst = inv_p.astype(jnp.int32).reshape(NP // 128, 128)
    accp = _sc_spmm(hs_p, pool_src, pool_dst, zrow, NP // 128)
    h2_p = accp[0] + accp[1]

    src2 = inv1[src].astype(jnp.int32)
    dst2 = inv1[dst].astype(jnp.int32)
    src2_c = src2.reshape(E // 128, 128)
    dst2_c = dst2.reshape(E // 128, 128)

    dis2, iv2 = level_norm(dst2)
    h = _gcn_conv_sc(h2_p, src2_c, dst2_c, zrow, dis2, iv2, W3, b3)
    h = _gcn_conv_sc(h, src2_c, dst2_c, zrow, dis2, iv2, W4, b4)

    hr = h[:N]
    ei2 = jnp.stack([src2, dst2])
    s2 = _edge_scores(hr, ei2, pw2, pb2)
    mask2, inv2 = _cluster_structure(s2, ei2, N)
    valid = jnp.zeros((N,), jnp.int32).at[inv2].max(valid)

    sm2 = jnp.where(mask2, s2, NEG)
    ns2 = jnp.full((N,), NEG, jnp.float32).at[src2].max(sm2).at[dst2].max(sm2)
    ns2 = jnp.where(ns2 <= NEG * 0.5, 1.0, ns2)
    hs2 = hr * ns2[:, None]
    hs2_p = jnp.pad(hs2, ((0, NP - N), (0, 0)))
    inv2_p = jnp.concatenate([inv2, jnp.arange(N, NP, dtype=jnp.int32)])
    pool_dst2 = inv2_p.astype(jnp.int32).reshape(NP // 128, 128)
    accp2 = _sc_spmm(hs2_p, pool_src, pool_dst2, zrow, NP // 128)
    h3_p = accp2[0] + accp2[1]

    src3 = inv2[src2].astype(jnp.int32)
    dst3 = inv2[dst2].astype(jnp.int32)
    src3_c = src3.reshape(E // 128, 128)
    dst3_c = dst3.reshape(E // 128, 128)

    dis3, iv3 = level_norm(dst3)
    h = _gcn_conv_sc(h3_p, src3_c, dst3_c, zrow, dis3, iv3, W5, b5)

    hr = h[:N]
    vf = valid.astype(jnp.float32)
    gsum = jnp.sum(hr * vf[:, None], axis=0, keepdims=True)
    cnt = jnp.sum(vf)
    g = gsum / jnp.maximum(cnt, 1.0)
    g = jax.nn.relu(_mm(g, fw1, bm=8) + fb1)
    out = jax.nn.sigmoid(_mm(g, fw2, bm=8) + fb2)
    return out.reshape(-1)


# speedup vs baseline: 4.3901x; 3.5454x over previous
import functools

import jax
import jax.numpy as jnp
from jax import lax
from jax.experimental import pallas as pl
from jax.experimental.pallas import tpu as pltpu
from jax.experimental.pallas import tpu_sc as plsc

N = 10000
NP = 10240
NPA = NP + 128
E = 160000
EC = E // 128
D = 128
NEG = -1e30
BIG = 0x7F7F7F7F
_STRIPE = NP // 16
_ASTRIPE = NPA // 16


def _mm_body(x_ref, w_ref, b_ref, o_ref):
    o_ref[...] = jnp.dot(x_ref[...], w_ref[...],
                         preferred_element_type=jnp.float32) + b_ref[...]


def _mm(x, w, bias, bm=1024):
    k = x.shape[1]
    ncp = w.shape[1]
    return pl.pallas_call(
        _mm_body,
        grid=(NP // bm,),
        in_specs=[pl.BlockSpec((bm, k), lambda i: (i, 0)),
                  pl.BlockSpec((k, ncp), lambda i: (0, 0)),
                  pl.BlockSpec((1, ncp), lambda i: (0, 0))],
        out_specs=pl.BlockSpec((bm, ncp), lambda i: (i, 0)),
        out_shape=jax.ShapeDtypeStruct((NP, ncp), jnp.float32),
    )(x, w, bias)


def _mm_conv_body(x_ref, w_ref, dis_ref, xw_ref, xws_ref):
    t = jnp.dot(x_ref[...], w_ref[...], preferred_element_type=jnp.float32)
    xw_ref[...] = t
    xws_ref[...] = t * dis_ref[...]


def _mm_conv(x, w, dis_col, bm=1024):
    k = x.shape[1]
    return pl.pallas_call(
        _mm_conv_body,
        grid=(NP // bm,),
        in_specs=[pl.BlockSpec((bm, k), lambda i: (i, 0)),
                  pl.BlockSpec((k, D), lambda i: (0, 0)),
                  pl.BlockSpec((bm, 1), lambda i: (i, 0))],
        out_specs=[pl.BlockSpec((bm, D), lambda i: (i, 0)),
                   pl.BlockSpec((bm, D), lambda i: (i, 0))],
        out_shape=[jax.ShapeDtypeStruct((NP, D), jnp.float32),
                   jax.ShapeDtypeStruct((NP, D), jnp.float32)],
    )(x, w, dis_col)


def _elt_post_body(a_ref, b_ref, xw_ref, dis_ref, iv_ref, bias_ref, o_ref):
    agg = (a_ref[...] + b_ref[...]) * dis_ref[...]
    o_ref[...] = jnp.maximum(agg + xw_ref[...] * iv_ref[...] + bias_ref[...],
                             0.0)


def _elt_post(acc2, xw, dis_col, iveff_col, bias, bm=1024):
    return pl.pallas_call(
        _elt_post_body,
        grid=(NP // bm,),
        in_specs=[pl.BlockSpec((bm, D), lambda i: (i, 0)),
                  pl.BlockSpec((bm, D), lambda i: (i, 0)),
                  pl.BlockSpec((bm, D), lambda i: (i, 0)),
                  pl.BlockSpec((bm, 1), lambda i: (i, 0)),
                  pl.BlockSpec((bm, 1), lambda i: (i, 0)),
                  pl.BlockSpec((1, D), lambda i: (0, 0))],
        out_specs=pl.BlockSpec((bm, D), lambda i: (i, 0)),
        out_shape=jax.ShapeDtypeStruct((NP, D), jnp.float32),
    )(acc2[0], acc2[1], xw, dis_col, iveff_col, bias.reshape(1, D))


def _add2_body(a_ref, b_ref, o_ref):
    o_ref[...] = a_ref[...] + b_ref[...]


def _add2(a, b, bm=1024):
    return pl.pallas_call(
        _add2_body,
        grid=(NP // bm,),
        in_specs=[pl.BlockSpec((bm, D), lambda i: (i, 0)),
                  pl.BlockSpec((bm, D), lambda i: (i, 0))],
        out_specs=pl.BlockSpec((bm, D), lambda i: (i, 0)),
        out_shape=jax.ShapeDtypeStruct((NP, D), jnp.float32),
    )(a, b)


def _norms_body(da_ref, db_ref, sa_ref, sb_ref, dis_ref, iv_ref):
    deg = da_ref[...] + db_ref[...] + 1.0
    selfc = sa_ref[...] + sb_ref[...]
    dis = lax.rsqrt(deg)
    dis_ref[...] = dis
    iv_ref[...] = (1.0 + selfc) / deg


def _norms(deg2, self2):
    outs = pl.pallas_call(
        _norms_body,
        in_specs=[pl.BlockSpec((80, 128), lambda: (0, 0))] * 4,
        out_specs=[pl.BlockSpec((80, 128), lambda: (0, 0))] * 2,
        out_shape=[jax.ShapeDtypeStruct((80, 128), jnp.float32)] * 2,
    )(deg2[0].reshape(80, 128), deg2[1].reshape(80, 128),
      self2[0].reshape(80, 128), self2[1].reshape(80, 128))
    return outs[0].reshape(NP, 1), outs[1].reshape(NP, 1)


def _scale_body(h_ref, na_ref, nb_ref, o_ref):
    ns = jnp.maximum(na_ref[...], nb_ref[...])
    ns = jnp.where(ns <= NEG * 0.5, 1.0, ns)
    o_ref[...] = h_ref[...] * ns


def _scale_hs(h, ns2, bm=1024):
    return pl.pallas_call(
        _scale_body,
        grid=(NP // bm,),
        in_specs=[pl.BlockSpec((bm, D), lambda i: (i, 0)),
                  pl.BlockSpec((bm, 1), lambda i: (i, 0)),
                  pl.BlockSpec((bm, 1), lambda i: (i, 0))],
        out_specs=pl.BlockSpec((bm, D), lambda i: (i, 0)),
        out_shape=jax.ShapeDtypeStruct((NP, D), jnp.float32),
    )(h, ns2[0].reshape(NP, 1), ns2[1].reshape(NP, 1))


def _head_body(h_ref, v_ref, fw1_ref, fb1_ref, fw2_ref, fb2_ref, o_ref,
               acc_ref):
    p = pl.program_id(0)
    bm = h_ref.shape[0]

    @pl.when(p == 0)
    def _():
        acc_ref[...] = jnp.zeros_like(acc_ref)

    rows = lax.broadcasted_iota(jnp.int32, (bm, 1), 0) + p * bm
    vf = jnp.where(rows < N, v_ref[...], 0.0)
    hv = h_ref[...] * vf
    acc_ref[0:1, :] += jnp.sum(hv, axis=0, keepdims=True)
    acc_ref[1:2, :] += jnp.sum(vf)

    @pl.when(p == pl.num_programs(0) - 1)
    def _():
        cnt = acc_ref[1, 0]
        g = acc_ref[0:1, :] / jnp.maximum(cnt, 1.0)
        z = jnp.maximum(
            jnp.dot(g, fw1_ref[...], preferred_element_type=jnp.float32)
            + fb1_ref[...], 0.0)
        o = jnp.dot(z, fw2_ref[...], preferred_element_type=jnp.float32)
        o = 1.0 / (1.0 + jnp.exp(-(o + fb2_ref[...])))
        o_ref[...] = o


def _head(h, valid_col, fw1, fb1, fw2p, fb2p, bm=1024):
    return pl.pallas_call(
        _head_body,
        grid=(NP // bm,),
        in_specs=[pl.BlockSpec((bm, D), lambda i: (i, 0)),
                  pl.BlockSpec((bm, 1), lambda i: (i, 0)),
                  pl.BlockSpec((D, D), lambda i: (0, 0)),
                  pl.BlockSpec((1, D), lambda i: (0, 0)),
                  pl.BlockSpec((D, D), lambda i: (0, 0)),
                  pl.BlockSpec((1, D), lambda i: (0, 0))],
        out_specs=pl.BlockSpec((1, D), lambda i: (0, 0)),
        out_shape=jax.ShapeDtypeStruct((1, D), jnp.float32),
        scratch_shapes=[pltpu.VMEM((8, D), jnp.float32)],
    )(h, valid_col, fw1, fb1, fw2p, fb2p)


_MESH = dict(core_axis_name="c", subcore_axis_name="s")


def _iota16():
    return lax.iota(jnp.int32, 16)


def _shift(x, sh, fill):
    it = _iota16()
    idx = jnp.minimum(it + sh, 15)
    g = x.at[idx].get(mode="promise_in_bounds")
    return jnp.where(it < 16 - sh, g, fill)


def _seg_reduce(keys, vals, neutral, op):
    k, v = plsc.sort_key_val(keys, vals)
    for sh in (1, 2, 4, 8):
        k_sh = _shift(k, sh, jnp.int32(-1))
        v_sh = _shift(v, sh, neutral)
        v = jnp.where(k_sh == k, op(v, v_sh), v)
    it = _iota16()
    kp = k.at[jnp.maximum(it - 1, 0)].get(mode="promise_in_bounds")
    head = (it == 0) | (kp != k)
    return k, v, head


def _chunk_range(half, sid):
    start = (sid * half) // 16
    cnt = ((sid + 1) * half) // 16 - start
    return start, cnt


def _combine_stripe(stage, colblk, res, out_ref, cid, sid, op, neutral):
    base = sid * _STRIPE
    pltpu.sync_copy(stage.at[:, pl.ds(base, _STRIPE)], colblk)

    def body(g, carry):
        a = colblk[0, pl.ds(g * 16, 16)]
        for r in range(1, 16):
            a = op(a, colblk[r, pl.ds(g * 16, 16)])
        res[pl.ds(g * 16, 16)] = a
        return carry

    lax.fori_loop(0, _STRIPE // 16, body, 0)
    if out_ref is not None:
        pltpu.sync_copy(res, out_ref.at[pl.ds(cid * NP + base, _STRIPE)])


def _spmm_body(nchunks, xs_ref, src_ref, dst_ref, zrow_ref, out_ref,
               sidx, didx, rows0, rows1, acc, s0, s1, semz):
    cid = lax.axis_index("c")
    sid = lax.axis_index("s")
    half = nchunks // 2
    pre = (half + 15) // 16
    astart = sid * _ASTRIPE

    for i in range(5):
        pltpu.async_copy(zrow_ref, acc.at[pl.ds(astart + i * 128, 128)],
                         semz).wait()
    pltpu.async_copy(zrow_ref.at[pl.ds(0, 8)],
                     acc.at[pl.ds(astart + 640, 8)], semz).wait()

    start, cnt = _chunk_range(half, sid)
    start = start + cid * half
    pltpu.sync_copy(src_ref.at[pl.ds(start, pre)], sidx)
    pltpu.sync_copy(dst_ref.at[pl.ds(start, pre)], didx)
    plsc.subcore_barrier()

    def body(j, carry):
        pltpu.async_copy(xs_ref.at[sidx.at[j]], rows0, s0).wait()
        pltpu.sync_copy(rows0, acc.at[didx.at[j]], add=True)
        return carry

    lax.fori_loop(0, cnt, body, 0)
    plsc.subcore_barrier()
    pltpu.sync_copy(acc.at[pl.ds(astart, _ASTRIPE)],
                    out_ref.at[pl.ds(cid * NPA + astart, _ASTRIPE)])


def _sc_spmm(xs, src_c, dst_c, zrow):
    nchunks = src_c.shape[0]
    flat = pl.kernel(
        functools.partial(_spmm_body, nchunks),
        out_type=jax.ShapeDtypeStruct((2 * NPA, D), jnp.float32),
        mesh=plsc.VectorSubcoreMesh(**_MESH),
        compiler_params=pltpu.CompilerParams(use_tc_tiling_on_sc=False, needs_layout_passes=False),
        scratch_types=[
            pltpu.VMEM(((nchunks // 2 + 15) // 16, 128), jnp.int32),
            pltpu.VMEM(((nchunks // 2 + 15) // 16, 128), jnp.int32),
            pltpu.VMEM((128, D), jnp.float32),
            pltpu.VMEM((128, D), jnp.float32),
            pltpu.VMEM_SHARED((NPA, D), jnp.float32),
            pltpu.SemaphoreType.DMA,
            pltpu.SemaphoreType.DMA,
            pltpu.SemaphoreType.DMA,
        ],
    )(xs, src_c, dst_c, zrow)
    return flat.reshape(2, NPA, D)


def _deg_body(src_ref, dst_ref, zn_ref, deg_ref, self_ref, dstr_ref,
              se, de, drrow, deg_v, self_v, colblk, res, stage):
    cid = lax.axis_index("c")
    sid = lax.axis_index("s")
    half = EC // 2
    pre = (half + 15) // 16
    start, cnt = _chunk_range(half, sid)
    start = start + cid * half

    pltpu.sync_copy(src_ref.at[pl.ds(start, pre)], se)
    pltpu.sync_copy(dst_ref.at[pl.ds(start, pre)], de)
    pltpu.sync_copy(zn_ref, deg_v)
    pltpu.sync_copy(zn_ref, self_v)

    def chunk(j, carry):
        for g in range(8):
            s16 = se[j, pl.ds(g * 16, 16)]
            d16 = de[j, pl.ds(g * 16, 16)]
            isself = s16 == d16
            k, v, head = _seg_reduce(d16, jnp.ones((16,), jnp.float32),
                                     0.0, jnp.add)
            cur = plsc.load_gather(deg_v, [k])
            plsc.store_scatter(deg_v, [k], cur + v, mask=head)
            ks = jnp.where(isself, d16, BIG)
            k2, v2, head2 = _seg_reduce(
                ks, jnp.where(isself, 1.0, 0.0).astype(jnp.float32),
                0.0, jnp.add)
            m2 = head2 & (k2 != BIG)
            k2c = jnp.where(k2 == BIG, 0, k2)
            cur2 = plsc.load_gather(self_v, [k2c])
            plsc.store_scatter(self_v, [k2c], cur2 + v2, mask=m2)
            drrow[pl.ds(g * 16, 16)] = jnp.where(
                isself, NP + g * 16 + _iota16(), d16)
        pltpu.sync_copy(drrow, dstr_ref.at[start + j])
        return carry

    lax.fori_loop(0, cnt, chunk, 0)

    pltpu.sync_copy(deg_v, stage.at[sid])
    plsc.subcore_barrier()
    _combine_stripe(stage, colblk, res, deg_ref, cid, sid, jnp.add,
                    0.0)
    plsc.subcore_barrier()
    pltpu.sync_copy(self_v, stage.at[sid])
    plsc.subcore_barrier()
    _combine_stripe(stage, colblk, res, self_ref, cid, sid, jnp.add,
                    0.0)


def _sc_degself(src_c, dst_c, zn):
    pre = (EC // 2 + 15) // 16
    outs = pl.kernel(
        _deg_body,
        out_type=[jax.ShapeDtypeStruct((2 * NP,), jnp.float32),
                  jax.ShapeDtypeStruct((2 * NP,), jnp.float32),
                  jax.ShapeDtypeStruct((EC, 128), jnp.int32)],
        mesh=plsc.VectorSubcoreMesh(**_MESH),
        compiler_params=pltpu.CompilerParams(use_tc_tiling_on_sc=False, needs_layout_passes=False),
        scratch_types=[
            pltpu.VMEM((pre, 128), jnp.int32),
            pltpu.VMEM((pre, 128), jnp.int32),
            pltpu.VMEM((128,), jnp.int32),
            pltpu.VMEM((NP,), jnp.float32),
            pltpu.VMEM((NP,), jnp.float32),
            pltpu.VMEM((16, _STRIPE), jnp.float32),
            pltpu.VMEM((_STRIPE,), jnp.float32),
            pltpu.VMEM_SHARED((16, NP), jnp.float32),
        ],
    )(src_c, dst_c, zn)
    return outs[0].reshape(2, NP), outs[1].reshape(2, NP), outs[2]


def _scores_body(src_ref, dst_ref, a_ref, b_ref, nfill_ref, act_ref, ns_ref,
                 se, de, a_v, b_v, ns_v, actrow, colblk, res, stage):
    cid = lax.axis_index("c")
    sid = lax.axis_index("s")
    half = EC // 2
    pre = (half + 15) // 16
    start, cnt = _chunk_range(half, sid)
    start = start + cid * half

    pltpu.sync_copy(src_ref.at[pl.ds(start, pre)], se)
    pltpu.sync_copy(dst_ref.at[pl.ds(start, pre)], de)
    pltpu.sync_copy(a_ref, a_v)
    pltpu.sync_copy(b_ref, b_v)
    pltpu.sync_copy(nfill_ref, ns_v)

    def chunk(j, carry):
        for g in range(8):
            s16 = se[j, pl.ds(g * 16, 16)]
            d16 = de[j, pl.ds(g * 16, 16)]
            av = plsc.load_gather(a_v, [s16])
            bv = plsc.load_gather(b_v, [d16])
            logit = av + bv
            act16 = logit > 0.0
            sig = 1.0 / (1.0 + jnp.exp(-logit))
            sm = jnp.where(act16, sig, NEG)
            actrow[pl.ds(g * 16, 16)] = act16.astype(jnp.int32)
            for tgt in (s16, d16):
                k, v, head = _seg_reduce(tgt, sm, NEG,
                                         jnp.maximum)
                cur = plsc.load_gather(ns_v, [k])
                m = head & (v > cur)
                plsc.store_scatter(ns_v, [k], v, mask=m)
        pltpu.sync_copy(actrow, act_ref.at[start + j])
        return carry

    lax.fori_loop(0, cnt, chunk, 0)
    pltpu.sync_copy(ns_v, stage.at[sid])
    plsc.subcore_barrier()
    _combine_stripe(stage, colblk, res, ns_ref, cid, sid, jnp.maximum,
                    NEG)


def _sc_scores(src_c, dst_c, a, b, nfill):
    pre = (EC // 2 + 15) // 16
    outs = pl.kernel(
        _scores_body,
        out_type=[jax.ShapeDtypeStruct((EC, 128), jnp.int32),
                  jax.ShapeDtypeStruct((2 * NP,), jnp.float32)],
        mesh=plsc.VectorSubcoreMesh(**_MESH),
        compiler_params=pltpu.CompilerParams(use_tc_tiling_on_sc=False, needs_layout_passes=False),
        scratch_types=[
            pltpu.VMEM((pre, 128), jnp.int32),
            pltpu.VMEM((pre, 128), jnp.int32),
            pltpu.VMEM((NP,), jnp.float32),
            pltpu.VMEM((NP,), jnp.float32),
            pltpu.VMEM((NP,), jnp.float32),
            pltpu.VMEM((128,), jnp.int32),
            pltpu.VMEM((16, _STRIPE), jnp.float32),
            pltpu.VMEM((_STRIPE,), jnp.float32),
            pltpu.VMEM_SHARED((16, NP), jnp.float32),
        ],
    )(src_c, dst_c, a, b, nfill)
    return outs[0], outs[1].reshape(2, NP)


def _cc_body(src_ref, dst_ref, act_ref, lbl0_ref, maskv_ref, z8_ref,
             ones8_ref, lbl_out, vld_out,
             se, de, ae, lbl_v, mask_v, colblk, res, ones8_v, flv,
             stage, lbl_sh, flags, vld_s):
    cid = lax.axis_index("c")
    sid = lax.axis_index("s")

    @pl.when(cid == 0)
    def _():
        pre = (EC + 15) // 16
        start, cnt = _chunk_range(EC, sid)
        base = sid * _STRIPE

        pltpu.sync_copy(src_ref.at[pl.ds(start, pre)], se)
        pltpu.sync_copy(dst_ref.at[pl.ds(start, pre)], de)
        pltpu.sync_copy(act_ref.at[pl.ds(start, pre)], ae)
        pltpu.sync_copy(lbl0_ref, lbl_v)
        pltpu.sync_copy(maskv_ref.at[pl.ds(base, _STRIPE)], mask_v)
        pltpu.sync_copy(ones8_ref, ones8_v)

        def one_iter(carry):
            changed = jnp.zeros((16,), jnp.int32)

            def chunk(j, ch):
                for g in range(8):
                    s16 = se[j, pl.ds(g * 16, 16)]
                    d16 = de[j, pl.ds(g * 16, 16)]
                    a16 = ae[j, pl.ds(g * 16, 16)]
                    ls = plsc.load_gather(lbl_v, [s16])
                    ld = plsc.load_gather(lbl_v, [d16])
                    lm = jnp.where(a16 > 0, jnp.minimum(ls, ld), BIG)
                    for tgt in (s16, d16):
                        k, v, head = _seg_reduce(tgt, lm, BIG, jnp.minimum)
                        cur = plsc.load_gather(lbl_v, [k])
                        m = head & (v < cur)
                        plsc.store_scatter(lbl_v, [k], v, mask=m)
                        ch = ch | m.astype(jnp.int32)
                return ch

            changed = lax.fori_loop(0, cnt, chunk, changed)

            def halve(g, carry2):
                idx = lbl_v[pl.ds(base + g * 16, 16)]
                l2 = plsc.load_gather(lbl_v, [idx])
                lbl_v[pl.ds(base + g * 16, 16)] = l2
                return carry2

            lax.fori_loop(0, _STRIPE // 16, halve, 0)

            pltpu.sync_copy(lbl_v, stage.at[sid])
            chs = jnp.max(changed) + jnp.zeros((16,), jnp.int32)
            res[pl.ds(0, 16)] = chs
            pltpu.sync_copy(res.at[pl.ds(0, 16)], flags.at[sid])
            plsc.subcore_barrier()

            _combine_stripe(stage, colblk, res, None, cid, sid,
                            jnp.minimum, BIG)
            pltpu.sync_copy(res, lbl_sh.at[pl.ds(base, _STRIPE)])
            pltpu.sync_copy(flags, flv)
            m = flv[0, pl.ds(0, 16)]
            for r in range(1, 16):
                m = jnp.maximum(m, flv[r, pl.ds(0, 16)])
            allch = jnp.max(m)
            plsc.subcore_barrier()
            pltpu.sync_copy(lbl_sh, lbl_v)
            return allch

        lax.while_loop(lambda c: c > 0, lambda c: one_iter(c),
                       1)

        for i in range(5):
            pltpu.sync_copy(z8_ref, vld_s.at[pl.ds(base + i * 128, 128)])
        plsc.subcore_barrier()

        def vscat(g, carry):
            lbl16 = lbl_v[pl.ds(base + g * 16, 16)]
            m16 = mask_v[pl.ds(g * 16, 16)]
            tgt = jnp.where(m16 > 0, lbl16, NP + _iota16())
            pltpu.sync_copy(ones8_v, vld_s.at[tgt])
            return carry

        lax.fori_loop(0, _STRIPE // 16, vscat, 0)
        plsc.subcore_barrier()
        pltpu.sync_copy(lbl_v.at[pl.ds(base, _STRIPE)],
                        lbl_out.at[pl.ds(base, _STRIPE)])
        pltpu.sync_copy(vld_s.at[pl.ds(base, _STRIPE)],
                        vld_out.at[pl.ds(base, _STRIPE)])


def _sc_cc(src_c, dst_c, act_c, lbl0, maskv, z8, ones8):
    pre = (EC + 15) // 16
    outs = pl.kernel(
        _cc_body,
        out_type=[jax.ShapeDtypeStruct((NP,), jnp.int32),
                  jax.ShapeDtypeStruct((NP, 8), jnp.int32)],
        mesh=plsc.VectorSubcoreMesh(**_MESH),
        compiler_params=pltpu.CompilerParams(use_tc_tiling_on_sc=False, needs_layout_passes=False),
        scratch_types=[
            pltpu.VMEM((pre, 128), jnp.int32),
            pltpu.VMEM((pre, 128), jnp.int32),
            pltpu.VMEM((pre, 128), jnp.int32),
            pltpu.VMEM((NP,), jnp.int32),
            pltpu.VMEM((_STRIPE,), jnp.int32),
            pltpu.VMEM((16, _STRIPE), jnp.int32),
            pltpu.VMEM((_STRIPE,), jnp.int32),
            pltpu.VMEM((16, 8), jnp.int32),
            pltpu.VMEM((16, 16), jnp.int32),
            pltpu.VMEM_SHARED((16, NP), jnp.int32),
            pltpu.VMEM_SHARED((NP,), jnp.int32),
            pltpu.VMEM_SHARED((16, 16), jnp.int32),
            pltpu.VMEM_SHARED((NP + 16, 8), jnp.int32),
        ],
    )(src_c, dst_c, act_c, lbl0, maskv, z8, ones8)
    return outs[0], outs[1]


def _einv_body(src_ref, dst_ref, inv_ref, srcn_ref, dstn_ref,
               se, de, inv_v, rs, rd):
    cid = lax.axis_index("c")
    sid = lax.axis_index("s")
    half = EC // 2
    pre = (half + 15) // 16
    start, cnt = _chunk_range(half, sid)
    start = start + cid * half

    pltpu.sync_copy(src_ref.at[pl.ds(start, pre)], se)
    pltpu.sync_copy(dst_ref.at[pl.ds(start, pre)], de)
    pltpu.sync_copy(inv_ref, inv_v)

    def chunk(j, carry):
        for g in range(8):
            s16 = se[j, pl.ds(g * 16, 16)]
            d16 = de[j, pl.ds(g * 16, 16)]
            rs[pl.ds(g * 16, 16)] = plsc.load_gather(inv_v, [s16])
            rd[pl.ds(g * 16, 16)] = plsc.load_gather(inv_v, [d16])
        pltpu.sync_copy(rs, srcn_ref.at[start + j])
        pltpu.sync_copy(rd, dstn_ref.at[start + j])
        return carry

    lax.fori_loop(0, cnt, chunk, 0)


def _sc_einv(src_c, dst_c, inv):
    pre = (EC // 2 + 15) // 16
    return pl.kernel(
        _einv_body,
        out_type=[jax.ShapeDtypeStruct((EC, 128), jnp.int32),
                  jax.ShapeDtypeStruct((EC, 128), jnp.int32)],
        mesh=plsc.VectorSubcoreMesh(**_MESH),
        compiler_params=pltpu.CompilerParams(use_tc_tiling_on_sc=False, needs_layout_passes=False),
        scratch_types=[
            pltpu.VMEM((pre, 128), jnp.int32),
            pltpu.VMEM((pre, 128), jnp.int32),
            pltpu.VMEM((NP,), jnp.int32),
            pltpu.VMEM((128,), jnp.int32),
            pltpu.VMEM((128,), jnp.int32),
        ],
    )(src_c, dst_c, inv)


def kernel(x, edge_list, batch, W1, b1, W2, b2, W3, b3, W4, b4, W5, b5,
           pw1, pb1, pw2, pb2, fw1, fb1, fw2, fb2, y=None):
    src_c = edge_list[:, 0].astype(jnp.int32).reshape(EC, 128)
    dst_c = edge_list[:, 1].astype(jnp.int32).reshape(EC, 128)
    zrow = jnp.zeros((128, D), jnp.float32)
    zn = jnp.zeros((NP,), jnp.float32)
    nfill = jnp.full((NP,), NEG, jnp.float32)
    z8 = jnp.zeros((128, 8), jnp.int32)
    ones8 = jnp.ones((16, 8), jnp.int32)
    lbl0 = jnp.arange(NP, dtype=jnp.int32)
    onesNP = jnp.ones((NP,), jnp.int32)
    pool_src = lbl0.reshape(NP // 128, 128)
    x_p = jnp.pad(x, ((0, NP - N), (0, 0)))

    def level(h, sc, dc, Ws, pw, pb):
        deg2, self2, dstr = _sc_degself(sc, dc, zn)
        dis, iveff = _norms(deg2, self2)
        for (W, b) in Ws:
            xw, xws = _mm_conv(h, W, dis)
            acc2 = _sc_spmm(xws, sc, dstr, zrow)
            h = _elt_post(acc2[:, :NP, :], xw, dis, iveff, b)
        pwcat = jnp.pad(jnp.concatenate([pw[:D], pw[D:]], axis=1),
                        ((0, 0), (0, 126)))
        bias2 = jnp.pad(pb.reshape(1, 1), ((0, 0), (0, 127)))
        ab = _mm(h, pwcat, bias2)
        return h, ab[:, 0], ab[:, 1]

    h, a1, b1v = level(x_p, src_c, dst_c, [(W1, b1), (W2, b2)], pw1, pb1)
    act1, ns1 = _sc_scores(src_c, dst_c, a1, b1v, nfill)
    lbl1, vld1 = _sc_cc(src_c, dst_c, act1, lbl0, onesNP, z8, ones8)
    hs = _scale_hs(h, ns1)
    accp = _sc_spmm(hs, pool_src, lbl1.reshape(NP // 128, 128), zrow)
    h2 = _add2(accp[0, :NP, :], accp[1, :NP, :])
    src2_c, dst2_c = _sc_einv(src_c, dst_c, lbl1)

    h, a2, b2v = level(h2, src2_c, dst2_c, [(W3, b3), (W4, b4)], pw2, pb2)
    act2, ns2 = _sc_scores(src2_c, dst2_c, a2, b2v, nfill)
    lbl2, vld2 = _sc_cc(src2_c, dst2_c, act2, lbl0, vld1[:, 0], z8, ones8)
    hs2 = _scale_hs(h, ns2)
    accp2 = _sc_spmm(hs2, pool_src, lbl2.reshape(NP // 128, 128), zrow)
    h3 = _add2(accp2[0, :NP, :], accp2[1, :NP, :])
    src3_c, dst3_c = _sc_einv(src2_c, dst2_c, lbl2)

    deg2, self2, dstr3 = _sc_degself(src3_c, dst3_c, zn)
    dis3, iveff3 = _norms(deg2, self2)
    xw, xws = _mm_conv(h3, W5, dis3)
    acc2 = _sc_spmm(xws, src3_c, dstr3, zrow)
    h5 = _elt_post(acc2[:, :NP, :], xw, dis3, iveff3, b5)

    valid_col = vld2[:, 0].astype(jnp.float32).reshape(NP, 1)
    fw2p = jnp.pad(fw2, ((0, 0), (0, 127)))
    fb2p = jnp.pad(fb2.reshape(1, 1), ((0, 0), (0, 127)))
    out = _head(h5, valid_col, fw1, fb1.reshape(1, D), fw2p, fb2p)
    return out[0, 0].reshape(1)


# speedup vs baseline: 24.8286x; 5.6556x over previous
import functools

import jax
import jax.numpy as jnp
from jax import lax
from jax.experimental import pallas as pl
from jax.experimental.pallas import tpu as pltpu
from jax.experimental.pallas import tpu_sc as plsc

N = 10000
NP = 10240
NPA = NP + 2048
E = 160000
EC = E // 128
D = 128
NEG = -1e30
BIG = 0x7F7F7F7F
_STRIPE = NP // 16
_ASTRIPE = NPA // 16


def _mm_body(x_ref, w_ref, b_ref, o_ref):
    o_ref[...] = jnp.dot(x_ref[...], w_ref[...],
                         preferred_element_type=jnp.float32) + b_ref[...]


def _mm(x, w, bias, bm=1024):
    k = x.shape[1]
    ncp = w.shape[1]
    return pl.pallas_call(
        _mm_body,
        grid=(NP // bm,),
        in_specs=[pl.BlockSpec((bm, k), lambda i: (i, 0)),
                  pl.BlockSpec((k, ncp), lambda i: (0, 0)),
                  pl.BlockSpec((1, ncp), lambda i: (0, 0))],
        out_specs=pl.BlockSpec((bm, ncp), lambda i: (i, 0)),
        out_shape=jax.ShapeDtypeStruct((NP, ncp), jnp.float32),
    )(x, w, bias)


def _mm_conv_body(x_ref, w_ref, dis_ref, xw_ref, xws_ref):
    t = jnp.dot(x_ref[...], w_ref[...], preferred_element_type=jnp.float32)
    xw_ref[...] = t
    xws_ref[...] = t * dis_ref[...]


def _mm_conv(x, w, dis_col, bm=1024):
    k = x.shape[1]
    return pl.pallas_call(
        _mm_conv_body,
        grid=(NP // bm,),
        in_specs=[pl.BlockSpec((bm, k), lambda i: (i, 0)),
                  pl.BlockSpec((k, D), lambda i: (0, 0)),
                  pl.BlockSpec((bm, 1), lambda i: (i, 0))],
        out_specs=[pl.BlockSpec((bm, D), lambda i: (i, 0)),
                   pl.BlockSpec((bm, D), lambda i: (i, 0))],
        out_shape=[jax.ShapeDtypeStruct((NP, D), jnp.float32),
                   jax.ShapeDtypeStruct((NP, D), jnp.float32)],
    )(x, w, dis_col)


def _elt_post_body(a_ref, b_ref, xw_ref, dis_ref, iv_ref, bias_ref, o_ref):
    agg = (a_ref[...] + b_ref[...]) * dis_ref[...]
    o_ref[...] = jnp.maximum(agg + xw_ref[...] * iv_ref[...] + bias_ref[...],
                             0.0)


def _elt_post(acc2, xw, dis_col, iveff_col, bias, bm=1024):
    return pl.pallas_call(
        _elt_post_body,
        grid=(NP // bm,),
        in_specs=[pl.BlockSpec((bm, D), lambda i: (i, 0)),
                  pl.BlockSpec((bm, D), lambda i: (i, 0)),
                  pl.BlockSpec((bm, D), lambda i: (i, 0)),
                  pl.BlockSpec((bm, 1), lambda i: (i, 0)),
                  pl.BlockSpec((bm, 1), lambda i: (i, 0)),
                  pl.BlockSpec((1, D), lambda i: (0, 0))],
        out_specs=pl.BlockSpec((bm, D), lambda i: (i, 0)),
        out_shape=jax.ShapeDtypeStruct((NP, D), jnp.float32),
    )(acc2[0], acc2[1], xw, dis_col, iveff_col, bias.reshape(1, D))


def _add2_body(a_ref, b_ref, o_ref):
    o_ref[...] = a_ref[...] + b_ref[...]


def _add2(a, b, bm=1024):
    return pl.pallas_call(
        _add2_body,
        grid=(NP // bm,),
        in_specs=[pl.BlockSpec((bm, D), lambda i: (i, 0)),
                  pl.BlockSpec((bm, D), lambda i: (i, 0))],
        out_specs=pl.BlockSpec((bm, D), lambda i: (i, 0)),
        out_shape=jax.ShapeDtypeStruct((NP, D), jnp.float32),
    )(a, b)


def _norms_body(da_ref, db_ref, sa_ref, sb_ref, dis_ref, iv_ref):
    deg = da_ref[...] + db_ref[...] + 1.0
    selfc = sa_ref[...] + sb_ref[...]
    dis = lax.rsqrt(deg)
    dis_ref[...] = dis
    iv_ref[...] = (1.0 + selfc) / deg


def _norms(deg2, self2):
    outs = pl.pallas_call(
        _norms_body,
        in_specs=[pl.BlockSpec((80, 128), lambda: (0, 0))] * 4,
        out_specs=[pl.BlockSpec((80, 128), lambda: (0, 0))] * 2,
        out_shape=[jax.ShapeDtypeStruct((80, 128), jnp.float32)] * 2,
    )(deg2[0].reshape(80, 128), deg2[1].reshape(80, 128),
      self2[0].reshape(80, 128), self2[1].reshape(80, 128))
    return outs[0].reshape(NP, 1), outs[1].reshape(NP, 1)


def _scale_body(h_ref, na_ref, nb_ref, o_ref):
    ns = jnp.maximum(na_ref[...], nb_ref[...])
    ns = jnp.where(ns <= NEG * 0.5, 1.0, ns)
    o_ref[...] = h_ref[...] * ns


def _scale_hs(h, ns2, bm=1024):
    return pl.pallas_call(
        _scale_body,
        grid=(NP // bm,),
        in_specs=[pl.BlockSpec((bm, D), lambda i: (i, 0)),
                  pl.BlockSpec((bm, 1), lambda i: (i, 0)),
                  pl.BlockSpec((bm, 1), lambda i: (i, 0))],
        out_specs=pl.BlockSpec((bm, D), lambda i: (i, 0)),
        out_shape=jax.ShapeDtypeStruct((NP, D), jnp.float32),
    )(h, ns2[0].reshape(NP, 1), ns2[1].reshape(NP, 1))


def _head_body(h_ref, v_ref, fw1_ref, fb1_ref, fw2_ref, fb2_ref, o_ref,
               acc_ref):
    p = pl.program_id(0)
    bm = h_ref.shape[0]

    @pl.when(p == 0)
    def _():
        acc_ref[...] = jnp.zeros_like(acc_ref)

    rows = lax.broadcasted_iota(jnp.int32, (bm, 1), 0) + p * bm
    vf = jnp.where(rows < N, v_ref[...], 0.0)
    hv = h_ref[...] * vf
    acc_ref[0:1, :] += jnp.sum(hv, axis=0, keepdims=True)
    acc_ref[1:2, :] += jnp.sum(vf)

    @pl.when(p == pl.num_programs(0) - 1)
    def _():
        cnt = acc_ref[1, 0]
        g = acc_ref[0:1, :] / jnp.maximum(cnt, 1.0)
        z = jnp.maximum(
            jnp.dot(g, fw1_ref[...], preferred_element_type=jnp.float32)
            + fb1_ref[...], 0.0)
        o = jnp.dot(z, fw2_ref[...], preferred_element_type=jnp.float32)
        o = 1.0 / (1.0 + jnp.exp(-(o + fb2_ref[...])))
        o_ref[...] = o


def _head(h, valid_col, fw1, fb1, fw2p, fb2p, bm=1024):
    return pl.pallas_call(
        _head_body,
        grid=(NP // bm,),
        in_specs=[pl.BlockSpec((bm, D), lambda i: (i, 0)),
                  pl.BlockSpec((bm, 1), lambda i: (i, 0)),
                  pl.BlockSpec((D, D), lambda i: (0, 0)),
                  pl.BlockSpec((1, D), lambda i: (0, 0)),
                  pl.BlockSpec((D, D), lambda i: (0, 0)),
                  pl.BlockSpec((1, D), lambda i: (0, 0))],
        out_specs=pl.BlockSpec((1, D), lambda i: (0, 0)),
        out_shape=jax.ShapeDtypeStruct((1, D), jnp.float32),
        scratch_shapes=[pltpu.VMEM((8, D), jnp.float32)],
    )(h, valid_col, fw1, fb1, fw2p, fb2p)


_MESH = dict(core_axis_name="c", subcore_axis_name="s")


def _iota16():
    return lax.iota(jnp.int32, 16)


def _shift(x, sh, fill):
    it = _iota16()
    idx = jnp.minimum(it + sh, 15)
    g = x.at[idx].get(mode="promise_in_bounds")
    return jnp.where(it < 16 - sh, g, fill)


def _seg_reduce(keys, vals, neutral, op):
    k, v = plsc.sort_key_val(keys, vals)
    for sh in (1, 2, 4, 8):
        k_sh = _shift(k, sh, jnp.int32(-1))
        v_sh = _shift(v, sh, neutral)
        v = jnp.where(k_sh == k, op(v, v_sh), v)
    it = _iota16()
    kp = k.at[jnp.maximum(it - 1, 0)].get(mode="promise_in_bounds")
    head = (it == 0) | (kp != k)
    return k, v, head


def _chunk_range(half, sid):
    start = (sid * half) // 16
    cnt = ((sid + 1) * half) // 16 - start
    return start, cnt


def _combine_stripe(stage, colblk, res, out_ref, cid, sid, op, neutral):
    base = sid * _STRIPE
    pltpu.sync_copy(stage.at[:, pl.ds(base, _STRIPE)], colblk)

    def body(g, carry):
        a = colblk[0, pl.ds(g * 16, 16)]
        for r in range(1, 16):
            a = op(a, colblk[r, pl.ds(g * 16, 16)])
        res[pl.ds(g * 16, 16)] = a
        return carry

    lax.fori_loop(0, _STRIPE // 16, body, 0)
    if out_ref is not None:
        pltpu.sync_copy(res, out_ref.at[pl.ds(cid * NP + base, _STRIPE)])


def _spmm_body(nchunks, xs_ref, src_ref, dst_ref, zrow_ref, out_ref,
               sidx, didx, rows0, rows1, acc, s0, s1, semz):
    cid = lax.axis_index("c")
    sid = lax.axis_index("s")
    half = nchunks // 2
    pre = (half + 15) // 16
    astart = sid * _ASTRIPE

    for i in range(6):
        pltpu.async_copy(zrow_ref, acc.at[pl.ds(astart + i * 128, 128)],
                         semz).wait()

    start, cnt = _chunk_range(half, sid)
    start = start + cid * half
    pltpu.sync_copy(src_ref.at[pl.ds(start, pre)], sidx)
    pltpu.sync_copy(dst_ref.at[pl.ds(start, pre)], didx)
    plsc.subcore_barrier()

    def body(j, carry):
        pltpu.async_copy(xs_ref.at[sidx.at[j]], rows0, s0).wait()
        pltpu.sync_copy(rows0, acc.at[didx.at[j]], add=True)
        return carry

    lax.fori_loop(0, cnt, body, 0)
    plsc.subcore_barrier()
    pltpu.sync_copy(acc.at[pl.ds(sid * _STRIPE, _STRIPE)],
                    out_ref.at[pl.ds(cid * NP + sid * _STRIPE, _STRIPE)])


def _sc_spmm(xs, src_c, dst_c, zrow):
    nchunks = src_c.shape[0]
    flat = pl.kernel(
        functools.partial(_spmm_body, nchunks),
        out_type=jax.ShapeDtypeStruct((2 * NP, D), jnp.float32),
        mesh=plsc.VectorSubcoreMesh(**_MESH),
        compiler_params=pltpu.CompilerParams(use_tc_tiling_on_sc=False, needs_layout_passes=False),
        scratch_types=[
            pltpu.VMEM(((nchunks // 2 + 15) // 16, 128), jnp.int32),
            pltpu.VMEM(((nchunks // 2 + 15) // 16, 128), jnp.int32),
            pltpu.VMEM((128, D), jnp.float32),
            pltpu.VMEM((128, D), jnp.float32),
            pltpu.VMEM_SHARED((NPA, D), jnp.float32),
            pltpu.SemaphoreType.DMA,
            pltpu.SemaphoreType.DMA,
            pltpu.SemaphoreType.DMA,
        ],
    )(xs, src_c, dst_c, zrow)
    return flat.reshape(2, NP, D)


def _deg_body(src_ref, dst_ref, zn_ref, deg_ref, self_ref, srcr_ref, dstr_ref,
              se, de, srrow, drrow, deg_v, self_v, colblk, res, stage):
    cid = lax.axis_index("c")
    sid = lax.axis_index("s")
    half = EC // 2
    pre = (half + 15) // 16
    start, cnt = _chunk_range(half, sid)
    start = start + cid * half

    pltpu.sync_copy(src_ref.at[pl.ds(start, pre)], se)
    pltpu.sync_copy(dst_ref.at[pl.ds(start, pre)], de)
    pltpu.sync_copy(zn_ref, deg_v)
    pltpu.sync_copy(zn_ref, self_v)

    def chunk(j, carry):
        for g in range(8):
            s16 = se[j, pl.ds(g * 16, 16)]
            d16 = de[j, pl.ds(g * 16, 16)]
            isself = s16 == d16
            k, v, head = _seg_reduce(d16, jnp.ones((16,), jnp.float32),
                                     0.0, jnp.add)
            cur = plsc.load_gather(deg_v, [k])
            plsc.store_scatter(deg_v, [k], cur + v, mask=head)
            ks = jnp.where(isself, d16, BIG)
            k2, v2, head2 = _seg_reduce(
                ks, jnp.where(isself, 1.0, 0.0).astype(jnp.float32),
                0.0, jnp.add)
            m2 = head2 & (k2 != BIG)
            k2c = jnp.where(k2 == BIG, 0, k2)
            cur2 = plsc.load_gather(self_v, [k2c])
            plsc.store_scatter(self_v, [k2c], cur2 + v2, mask=m2)
            pos = g * 16 + _iota16()
            drrow[pl.ds(g * 16, 16)] = jnp.where(
                isself, NP + sid * 128 + pos, d16)
            srrow[pl.ds(g * 16, 16)] = jnp.where(isself, j * 128 + pos, s16)
        pltpu.sync_copy(srrow, srcr_ref.at[start + j])
        pltpu.sync_copy(drrow, dstr_ref.at[start + j])
        return carry

    lax.fori_loop(0, cnt, chunk, 0)

    pltpu.sync_copy(deg_v, stage.at[sid])
    plsc.subcore_barrier()
    _combine_stripe(stage, colblk, res, deg_ref, cid, sid, jnp.add,
                    0.0)
    plsc.subcore_barrier()
    pltpu.sync_copy(self_v, stage.at[sid])
    plsc.subcore_barrier()
    _combine_stripe(stage, colblk, res, self_ref, cid, sid, jnp.add,
                    0.0)


def _sc_degself(src_c, dst_c, zn):
    pre = (EC // 2 + 15) // 16
    outs = pl.kernel(
        _deg_body,
        out_type=[jax.ShapeDtypeStruct((2 * NP,), jnp.float32),
                  jax.ShapeDtypeStruct((2 * NP,), jnp.float32),
                  jax.ShapeDtypeStruct((EC, 128), jnp.int32),
                  jax.ShapeDtypeStruct((EC, 128), jnp.int32)],
        mesh=plsc.VectorSubcoreMesh(**_MESH),
        compiler_params=pltpu.CompilerParams(use_tc_tiling_on_sc=False, needs_layout_passes=False),
        scratch_types=[
            pltpu.VMEM((pre, 128), jnp.int32),
            pltpu.VMEM((pre, 128), jnp.int32),
            pltpu.VMEM((128,), jnp.int32),
            pltpu.VMEM((128,), jnp.int32),
            pltpu.VMEM((NP,), jnp.float32),
            pltpu.VMEM((NP,), jnp.float32),
            pltpu.VMEM((16, _STRIPE), jnp.float32),
            pltpu.VMEM((_STRIPE,), jnp.float32),
            pltpu.VMEM_SHARED((16, NP), jnp.float32),
        ],
    )(src_c, dst_c, zn)
    return outs[0].reshape(2, NP), outs[1].reshape(2, NP), outs[2], outs[3]


def _scores_body(src_ref, dst_ref, a_ref, b_ref, nfill_ref, act_ref, ns_ref,
                 se, de, a_v, b_v, ns_v, actrow, colblk, res, stage):
    cid = lax.axis_index("c")
    sid = lax.axis_index("s")
    half = EC // 2
    pre = (half + 15) // 16
    start, cnt = _chunk_range(half, sid)
    start = start + cid * half

    pltpu.sync_copy(src_ref.at[pl.ds(start, pre)], se)
    pltpu.sync_copy(dst_ref.at[pl.ds(start, pre)], de)
    pltpu.sync_copy(a_ref, a_v)
    pltpu.sync_copy(b_ref, b_v)
    pltpu.sync_copy(nfill_ref, ns_v)

    def chunk(j, carry):
        for g in range(8):
            s16 = se[j, pl.ds(g * 16, 16)]
            d16 = de[j, pl.ds(g * 16, 16)]
            av = plsc.load_gather(a_v, [s16])
            bv = plsc.load_gather(b_v, [d16])
            logit = av + bv
            act16 = logit > 0.0
            sig = 1.0 / (1.0 + jnp.exp(-logit))
            sm = jnp.where(act16, sig, NEG)
            actrow[pl.ds(g * 16, 16)] = act16.astype(jnp.int32)
            for tgt in (s16, d16):
                k, v, head = _seg_reduce(tgt, sm, NEG,
                                         jnp.maximum)
                cur = plsc.load_gather(ns_v, [k])
                m = head & (v > cur)
                plsc.store_scatter(ns_v, [k], v, mask=m)
        pltpu.sync_copy(actrow, act_ref.at[start + j])
        return carry

    lax.fori_loop(0, cnt, chunk, 0)
    pltpu.sync_copy(ns_v, stage.at[sid])
    plsc.subcore_barrier()
    _combine_stripe(stage, colblk, res, ns_ref, cid, sid, jnp.maximum,
                    NEG)


def _sc_scores(src_c, dst_c, a, b, nfill):
    pre = (EC // 2 + 15) // 16
    outs = pl.kernel(
        _scores_body,
        out_type=[jax.ShapeDtypeStruct((EC, 128), jnp.int32),
                  jax.ShapeDtypeStruct((2 * NP,), jnp.float32)],
        mesh=plsc.VectorSubcoreMesh(**_MESH),
        compiler_params=pltpu.CompilerParams(use_tc_tiling_on_sc=False, needs_layout_passes=False),
        scratch_types=[
            pltpu.VMEM((pre, 128), jnp.int32),
            pltpu.VMEM((pre, 128), jnp.int32),
            pltpu.VMEM((NP,), jnp.float32),
            pltpu.VMEM((NP,), jnp.float32),
            pltpu.VMEM((NP,), jnp.float32),
            pltpu.VMEM((128,), jnp.int32),
            pltpu.VMEM((16, _STRIPE), jnp.float32),
            pltpu.VMEM((_STRIPE,), jnp.float32),
            pltpu.VMEM_SHARED((16, NP), jnp.float32),
        ],
    )(src_c, dst_c, a, b, nfill)
    return outs[0], outs[1].reshape(2, NP)


def _cc_body(src_ref, dst_ref, act_ref, lbl0_ref, maskv_ref, z8_ref,
             ones8_ref, lbl_out, vld_out,
             se, de, ae, lbl_v, mask_v, colblk, res, ones8_v, flv,
             stage, lbl_sh, flags, vld_s):
    cid = lax.axis_index("c")
    sid = lax.axis_index("s")

    @pl.when(cid == 0)
    def _():
        pre = (EC + 15) // 16
        start, cnt = _chunk_range(EC, sid)
        base = sid * _STRIPE

        pltpu.sync_copy(src_ref.at[pl.ds(start, pre)], se)
        pltpu.sync_copy(dst_ref.at[pl.ds(start, pre)], de)
        pltpu.sync_copy(act_ref.at[pl.ds(start, pre)], ae)
        pltpu.sync_copy(lbl0_ref, lbl_v)
        pltpu.sync_copy(maskv_ref.at[pl.ds(base, _STRIPE)], mask_v)
        pltpu.sync_copy(ones8_ref, ones8_v)

        def one_iter(carry):
            changed = jnp.zeros((16,), jnp.int32)

            def chunk(j, ch):
                for g in range(8):
                    s16 = se[j, pl.ds(g * 16, 16)]
                    d16 = de[j, pl.ds(g * 16, 16)]
                    a16 = ae[j, pl.ds(g * 16, 16)]
                    ls = plsc.load_gather(lbl_v, [s16])
                    ld = plsc.load_gather(lbl_v, [d16])
                    lm = jnp.where(a16 > 0, jnp.minimum(ls, ld), BIG)
                    for tgt in (s16, d16):
                        k, v, head = _seg_reduce(tgt, lm, BIG, jnp.minimum)
                        cur = plsc.load_gather(lbl_v, [k])
                        m = head & (v < cur)
                        plsc.store_scatter(lbl_v, [k], v, mask=m)
                        ch = ch | m.astype(jnp.int32)
                return ch

            changed = lax.fori_loop(0, cnt, chunk, changed)

            def halve(g, carry2):
                idx = lbl_v[pl.ds(base + g * 16, 16)]
                l2 = plsc.load_gather(lbl_v, [idx])
                lbl_v[pl.ds(base + g * 16, 16)] = l2
                return carry2

            lax.fori_loop(0, _STRIPE // 16, halve, 0)

            pltpu.sync_copy(lbl_v, stage.at[sid])
            chs = jnp.max(changed) + jnp.zeros((16,), jnp.int32)
            res[pl.ds(0, 16)] = chs
            pltpu.sync_copy(res.at[pl.ds(0, 16)], flags.at[sid])
            plsc.subcore_barrier()

            _combine_stripe(stage, colblk, res, None, cid, sid,
                            jnp.minimum, BIG)
            pltpu.sync_copy(res, lbl_sh.at[pl.ds(base, _STRIPE)])
            pltpu.sync_copy(flags, flv)
            m = flv[0, pl.ds(0, 16)]
            for r in range(1, 16):
                m = jnp.maximum(m, flv[r, pl.ds(0, 16)])
            allch = jnp.max(m)
            plsc.subcore_barrier()
            pltpu.sync_copy(lbl_sh, lbl_v)
            return allch

        lax.while_loop(lambda c: c > 0, lambda c: one_iter(c),
                       1)

        for i in range(5):
            pltpu.sync_copy(z8_ref, vld_s.at[pl.ds(base + i * 128, 128)])
        plsc.subcore_barrier()

        def vscat(g, carry):
            lbl16 = lbl_v[pl.ds(base + g * 16, 16)]
            m16 = mask_v[pl.ds(g * 16, 16)]
            tgt = jnp.where(m16 > 0, lbl16, NP + _iota16())
            pltpu.sync_copy(ones8_v, vld_s.at[tgt])
            return carry

        lax.fori_loop(0, _STRIPE // 16, vscat, 0)
        plsc.subcore_barrier()
        pltpu.sync_copy(lbl_v.at[pl.ds(base, _STRIPE)],
                        lbl_out.at[pl.ds(base, _STRIPE)])
        pltpu.sync_copy(vld_s.at[pl.ds(base, _STRIPE)],
                        vld_out.at[pl.ds(base, _STRIPE)])


def _sc_cc(src_c, dst_c, act_c, lbl0, maskv, z8, ones8):
    pre = (EC + 15) // 16
    outs = pl.kernel(
        _cc_body,
        out_type=[jax.ShapeDtypeStruct((NP,), jnp.int32),
                  jax.ShapeDtypeStruct((NP, 8), jnp.int32)],
        mesh=plsc.VectorSubcoreMesh(**_MESH),
        compiler_params=pltpu.CompilerParams(use_tc_tiling_on_sc=False, needs_layout_passes=False),
        scratch_types=[
            pltpu.VMEM((pre, 128), jnp.int32),
            pltpu.VMEM((pre, 128), jnp.int32),
            pltpu.VMEM((pre, 128), jnp.int32),
            pltpu.VMEM((NP,), jnp.int32),
            pltpu.VMEM((_STRIPE,), jnp.int32),
            pltpu.VMEM((16, _STRIPE), jnp.int32),
            pltpu.VMEM((_STRIPE,), jnp.int32),
            pltpu.VMEM((16, 8), jnp.int32),
            pltpu.VMEM((16, 16), jnp.int32),
            pltpu.VMEM_SHARED((16, NP), jnp.int32),
            pltpu.VMEM_SHARED((NP,), jnp.int32),
            pltpu.VMEM_SHARED((16, 16), jnp.int32),
            pltpu.VMEM_SHARED((NP + 16, 8), jnp.int32),
        ],
    )(src_c, dst_c, act_c, lbl0, maskv, z8, ones8)
    return outs[0], outs[1]


def _einv_body(src_ref, dst_ref, inv_ref, srcn_ref, dstn_ref,
               se, de, inv_v, rs, rd):
    cid = lax.axis_index("c")
    sid = lax.axis_index("s")
    half = EC // 2
    pre = (half + 15) // 16
    start, cnt = _chunk_range(half, sid)
    start = start + cid * half

    pltpu.sync_copy(src_ref.at[pl.ds(start, pre)], se)
    pltpu.sync_copy(dst_ref.at[pl.ds(start, pre)], de)
    pltpu.sync_copy(inv_ref, inv_v)

    def chunk(j, carry):
        for g in range(8):
            s16 = se[j, pl.ds(g * 16, 16)]
            d16 = de[j, pl.ds(g * 16, 16)]
            rs[pl.ds(g * 16, 16)] = plsc.load_gather(inv_v, [s16])
            rd[pl.ds(g * 16, 16)] = plsc.load_gather(inv_v, [d16])
        pltpu.sync_copy(rs, srcn_ref.at[start + j])
        pltpu.sync_copy(rd, dstn_ref.at[start + j])
        return carry

    lax.fori_loop(0, cnt, chunk, 0)


def _sc_einv(src_c, dst_c, inv):
    pre = (EC // 2 + 15) // 16
    return pl.kernel(
        _einv_body,
        out_type=[jax.ShapeDtypeStruct((EC, 128), jnp.int32),
                  jax.ShapeDtypeStruct((EC, 128), jnp.int32)],
        mesh=plsc.VectorSubcoreMesh(**_MESH),
        compiler_params=pltpu.CompilerParams(use_tc_tiling_on_sc=False, needs_layout_passes=False),
        scratch_types=[
            pltpu.VMEM((pre, 128), jnp.int32),
            pltpu.VMEM((pre, 128), jnp.int32),
            pltpu.VMEM((NP,), jnp.int32),
            pltpu.VMEM((128,), jnp.int32),
            pltpu.VMEM((128,), jnp.int32),
        ],
    )(src_c, dst_c, inv)


def kernel(x, edge_list, batch, W1, b1, W2, b2, W3, b3, W4, b4, W5, b5,
           pw1, pb1, pw2, pb2, fw1, fb1, fw2, fb2, y=None):
    src_c = edge_list[:, 0].astype(jnp.int32).reshape(EC, 128)
    dst_c = edge_list[:, 1].astype(jnp.int32).reshape(EC, 128)
    zrow = jnp.zeros((128, D), jnp.float32)
    zn = jnp.zeros((NP,), jnp.float32)
    nfill = jnp.full((NP,), NEG, jnp.float32)
    z8 = jnp.zeros((128, 8), jnp.int32)
    ones8 = jnp.ones((16, 8), jnp.int32)
    lbl0 = jnp.arange(NP, dtype=jnp.int32)
    onesNP = jnp.ones((NP,), jnp.int32)
    pool_src = lbl0.reshape(NP // 128, 128)
    x_p = jnp.pad(x, ((0, NP - N), (0, 0)))

    def level(h, sc, dc, Ws, pw, pb):
        deg2, self2, srcr, dstr = _sc_degself(sc, dc, zn)
        dis, iveff = _norms(deg2, self2)
        for (W, b) in Ws:
            xw, xws = _mm_conv(h, W, dis)
            acc2 = _sc_spmm(xws, srcr, dstr, zrow)
            h = _elt_post(acc2, xw, dis, iveff, b)
        pwcat = jnp.pad(jnp.concatenate([pw[:D], pw[D:]], axis=1),
                        ((0, 0), (0, 126)))
        bias2 = jnp.pad(pb.reshape(1, 1), ((0, 0), (0, 127)))
        ab = _mm(h, pwcat, bias2)
        return h, ab[:, 0], ab[:, 1]

    h, a1, b1v = level(x_p, src_c, dst_c, [(W1, b1), (W2, b2)], pw1, pb1)
    act1, ns1 = _sc_scores(src_c, dst_c, a1, b1v, nfill)
    lbl1, vld1 = _sc_cc(src_c, dst_c, act1, lbl0, onesNP, z8, ones8)
    hs = _scale_hs(h, ns1)
    accp = _sc_spmm(hs, pool_src, lbl1.reshape(NP // 128, 128), zrow)
    h2 = _add2(accp[0], accp[1])
    src2_c, dst2_c = _sc_einv(src_c, dst_c, lbl1)

    h, a2, b2v = level(h2, src2_c, dst2_c, [(W3, b3), (W4, b4)], pw2, pb2)
    act2, ns2 = _sc_scores(src2_c, dst2_c, a2, b2v, nfill)
    lbl2, vld2 = _sc_cc(src2_c, dst2_c, act2, lbl0, vld1[:, 0], z8, ones8)
    hs2 = _scale_hs(h, ns2)
    accp2 = _sc_spmm(hs2, pool_src, lbl2.reshape(NP // 128, 128), zrow)
    h3 = _add2(accp2[0], accp2[1])
    src3_c, dst3_c = _sc_einv(src2_c, dst2_c, lbl2)

    deg2, self2, srcr3, dstr3 = _sc_degself(src3_c, dst3_c, zn)
    dis3, iveff3 = _norms(deg2, self2)
    xw, xws = _mm_conv(h3, W5, dis3)
    acc2 = _sc_spmm(xws, srcr3, dstr3, zrow)
    h5 = _elt_post(acc2, xw, dis3, iveff3, b5)

    valid_col = vld2[:, 0].astype(jnp.float32).reshape(NP, 1)
    fw2p = jnp.pad(fw2, ((0, 0), (0, 127)))
    fb2p = jnp.pad(fb2.reshape(1, 1), ((0, 0), (0, 127)))
    out = _head(h5, valid_col, fw1, fb1.reshape(1, D), fw2p, fb2p)
    return out[0, 0].reshape(1)


# speedup vs baseline: 26.2360x; 1.0567x over previous
import functools

import jax
import jax.numpy as jnp
from jax import lax
from jax.experimental import pallas as pl
from jax.experimental.pallas import tpu as pltpu
from jax.experimental.pallas import tpu_sc as plsc

N = 10000
NP = 10240
NPA = NP + 256
E = 160000
EC = E // 128
D = 128
NEG = -1e30
BIG = 0x7F7F7F7F
_STRIPE = NP // 16
_ASTRIPE = NPA // 16


def _mm_body(x_ref, w_ref, b_ref, o_ref):
    o_ref[...] = jnp.dot(x_ref[...], w_ref[...],
                         preferred_element_type=jnp.float32) + b_ref[...]


def _mm(x, w, bias, bm=1024):
    k = x.shape[1]
    ncp = w.shape[1]
    return pl.pallas_call(
        _mm_body,
        grid=(NP // bm,),
        in_specs=[pl.BlockSpec((bm, k), lambda i: (i, 0)),
                  pl.BlockSpec((k, ncp), lambda i: (0, 0)),
                  pl.BlockSpec((1, ncp), lambda i: (0, 0))],
        out_specs=pl.BlockSpec((bm, ncp), lambda i: (i, 0)),
        out_shape=jax.ShapeDtypeStruct((NP, ncp), jnp.float32),
    )(x, w, bias)


def _mm_conv_body(x_ref, w_ref, dis_ref, xw_ref, xws_ref):
    t = jnp.dot(x_ref[...], w_ref[...], preferred_element_type=jnp.float32)
    xw_ref[...] = t
    xws_ref[...] = t * dis_ref[...]


def _mm_conv(x, w, dis_col, bm=1024):
    k = x.shape[1]
    return pl.pallas_call(
        _mm_conv_body,
        grid=(NP // bm,),
        in_specs=[pl.BlockSpec((bm, k), lambda i: (i, 0)),
                  pl.BlockSpec((k, D), lambda i: (0, 0)),
                  pl.BlockSpec((bm, 1), lambda i: (i, 0))],
        out_specs=[pl.BlockSpec((bm, D), lambda i: (i, 0)),
                   pl.BlockSpec((bm, D), lambda i: (i, 0))],
        out_shape=[jax.ShapeDtypeStruct((NP, D), jnp.float32),
                   jax.ShapeDtypeStruct((NP, D), jnp.float32)],
    )(x, w, dis_col)


def _elt_post_body(a_ref, b_ref, xw_ref, dis_ref, iv_ref, bias_ref, o_ref):
    agg = (a_ref[...] + b_ref[...]) * dis_ref[...]
    o_ref[...] = jnp.maximum(agg + xw_ref[...] * iv_ref[...] + bias_ref[...],
                             0.0)


def _elt_post(acc2, xw, dis_col, iveff_col, bias, bm=1024):
    return pl.pallas_call(
        _elt_post_body,
        grid=(NP // bm,),
        in_specs=[pl.BlockSpec((bm, D), lambda i: (i, 0)),
                  pl.BlockSpec((bm, D), lambda i: (i, 0)),
                  pl.BlockSpec((bm, D), lambda i: (i, 0)),
                  pl.BlockSpec((bm, 1), lambda i: (i, 0)),
                  pl.BlockSpec((bm, 1), lambda i: (i, 0)),
                  pl.BlockSpec((1, D), lambda i: (0, 0))],
        out_specs=pl.BlockSpec((bm, D), lambda i: (i, 0)),
        out_shape=jax.ShapeDtypeStruct((NP, D), jnp.float32),
    )(acc2[0], acc2[1], xw, dis_col, iveff_col, bias.reshape(1, D))


def _add2_body(a_ref, b_ref, o_ref):
    o_ref[...] = a_ref[...] + b_ref[...]


def _add2(a, b, bm=1024):
    return pl.pallas_call(
        _add2_body,
        grid=(NP // bm,),
        in_specs=[pl.BlockSpec((bm, D), lambda i: (i, 0)),
                  pl.BlockSpec((bm, D), lambda i: (i, 0))],
        out_specs=pl.BlockSpec((bm, D), lambda i: (i, 0)),
        out_shape=jax.ShapeDtypeStruct((NP, D), jnp.float32),
    )(a, b)


def _norms_body(da_ref, db_ref, sa_ref, sb_ref, dis_ref, iv_ref):
    deg = da_ref[...] + db_ref[...] + 1.0
    selfc = sa_ref[...] + sb_ref[...]
    dis = lax.rsqrt(deg)
    dis_ref[...] = dis
    iv_ref[...] = (1.0 + selfc) / deg


def _norms(deg2, self2):
    outs = pl.pallas_call(
        _norms_body,
        in_specs=[pl.BlockSpec((80, 128), lambda: (0, 0))] * 4,
        out_specs=[pl.BlockSpec((80, 128), lambda: (0, 0))] * 2,
        out_shape=[jax.ShapeDtypeStruct((80, 128), jnp.float32)] * 2,
    )(deg2[0].reshape(80, 128), deg2[1].reshape(80, 128),
      self2[0].reshape(80, 128), self2[1].reshape(80, 128))
    return outs[0].reshape(NP, 1), outs[1].reshape(NP, 1)


def _scale_body(h_ref, na_ref, nb_ref, o_ref):
    ns = jnp.maximum(na_ref[...], nb_ref[...])
    ns = jnp.where(ns <= NEG * 0.5, 1.0, ns)
    o_ref[...] = h_ref[...] * ns


def _scale_hs(h, ns2, bm=1024):
    return pl.pallas_call(
        _scale_body,
        grid=(NP // bm,),
        in_specs=[pl.BlockSpec((bm, D), lambda i: (i, 0)),
                  pl.BlockSpec((bm, 1), lambda i: (i, 0)),
                  pl.BlockSpec((bm, 1), lambda i: (i, 0))],
        out_specs=pl.BlockSpec((bm, D), lambda i: (i, 0)),
        out_shape=jax.ShapeDtypeStruct((NP, D), jnp.float32),
    )(h, ns2[0].reshape(NP, 1), ns2[1].reshape(NP, 1))


def _head_body(h_ref, v_ref, fw1_ref, fb1_ref, fw2_ref, fb2_ref, o_ref,
               acc_ref):
    p = pl.program_id(0)
    bm = h_ref.shape[0]

    @pl.when(p == 0)
    def _():
        acc_ref[...] = jnp.zeros_like(acc_ref)

    rows = lax.broadcasted_iota(jnp.int32, (bm, 1), 0) + p * bm
    vf = jnp.where(rows < N, v_ref[...], 0.0)
    hv = h_ref[...] * vf
    acc_ref[0:1, :] += jnp.sum(hv, axis=0, keepdims=True)
    acc_ref[1:2, :] += jnp.sum(vf)

    @pl.when(p == pl.num_programs(0) - 1)
    def _():
        cnt = acc_ref[1, 0]
        g = acc_ref[0:1, :] / jnp.maximum(cnt, 1.0)
        z = jnp.maximum(
            jnp.dot(g, fw1_ref[...], preferred_element_type=jnp.float32)
            + fb1_ref[...], 0.0)
        o = jnp.dot(z, fw2_ref[...], preferred_element_type=jnp.float32)
        o = 1.0 / (1.0 + jnp.exp(-(o + fb2_ref[...])))
        o_ref[...] = o


def _head(h, valid_col, fw1, fb1, fw2p, fb2p, bm=1024):
    return pl.pallas_call(
        _head_body,
        grid=(NP // bm,),
        in_specs=[pl.BlockSpec((bm, D), lambda i: (i, 0)),
                  pl.BlockSpec((bm, 1), lambda i: (i, 0)),
                  pl.BlockSpec((D, D), lambda i: (0, 0)),
                  pl.BlockSpec((1, D), lambda i: (0, 0)),
                  pl.BlockSpec((D, D), lambda i: (0, 0)),
                  pl.BlockSpec((1, D), lambda i: (0, 0))],
        out_specs=pl.BlockSpec((1, D), lambda i: (0, 0)),
        out_shape=jax.ShapeDtypeStruct((1, D), jnp.float32),
        scratch_shapes=[pltpu.VMEM((8, D), jnp.float32)],
    )(h, valid_col, fw1, fb1, fw2p, fb2p)


_MESH = dict(core_axis_name="c", subcore_axis_name="s")


def _iota16():
    return lax.iota(jnp.int32, 16)


def _shift(x, sh, fill):
    it = _iota16()
    idx = jnp.minimum(it + sh, 15)
    g = x.at[idx].get(mode="promise_in_bounds")
    return jnp.where(it < 16 - sh, g, fill)


def _seg_reduce(keys, vals, neutral, op):
    k, v = plsc.sort_key_val(keys, vals)
    for sh in (1, 2, 4, 8):
        k_sh = _shift(k, sh, jnp.int32(-1))
        v_sh = _shift(v, sh, neutral)
        v = jnp.where(k_sh == k, op(v, v_sh), v)
    it = _iota16()
    kp = k.at[jnp.maximum(it - 1, 0)].get(mode="promise_in_bounds")
    head = (it == 0) | (kp != k)
    return k, v, head


def _chunk_range(half, sid):
    start = (sid * half) // 16
    cnt = ((sid + 1) * half) // 16 - start
    return start, cnt


def _combine_stripe(stage, colblk, res, out_ref, cid, sid, op, neutral):
    base = sid * _STRIPE
    pltpu.sync_copy(stage.at[:, pl.ds(base, _STRIPE)], colblk)

    def body(g, carry):
        a = colblk[0, pl.ds(g * 16, 16)]
        for r in range(1, 16):
            a = op(a, colblk[r, pl.ds(g * 16, 16)])
        res[pl.ds(g * 16, 16)] = a
        return carry

    lax.fori_loop(0, _STRIPE // 16, body, 0)
    if out_ref is not None:
        pltpu.sync_copy(res, out_ref.at[pl.ds(cid * NP + base, _STRIPE)])


def _spmm_body(nchunks, xs_ref, src_ref, dst_ref, zrow_ref, out_ref,
               sidx, didx, rows0, rows1, acc, s0, s1, semz):
    cid = lax.axis_index("c")
    sid = lax.axis_index("s")
    half = nchunks // 2
    pre = (half + 15) // 16
    astart = sid * _ASTRIPE

    for i in range(5):
        pltpu.async_copy(zrow_ref, acc.at[pl.ds(astart + i * 128, 128)],
                         semz).wait()
    pltpu.async_copy(zrow_ref.at[pl.ds(0, 16)],
                     acc.at[pl.ds(astart + 640, 16)], semz).wait()

    start, cnt = _chunk_range(half, sid)
    start = start + cid * half
    pltpu.sync_copy(src_ref.at[pl.ds(start, pre)], sidx)
    pltpu.sync_copy(dst_ref.at[pl.ds(start, pre)], didx)
    plsc.subcore_barrier()

    def body(p, carry):
        j = 2 * p
        h0 = pltpu.async_copy(xs_ref.at[sidx.at[j]], rows0, s0)
        h1 = pltpu.async_copy(xs_ref.at[sidx.at[j + 1]], rows1, s1)
        h0.wait()
        pltpu.sync_copy(rows0, acc.at[didx.at[j]], add=True)
        h1.wait()
        pltpu.sync_copy(rows1, acc.at[didx.at[j + 1]], add=True)
        return carry

    lax.fori_loop(0, cnt // 2, body, 0)

    @pl.when(cnt % 2 == 1)
    def _():
        j = cnt - 1
        pltpu.async_copy(xs_ref.at[sidx.at[j]], rows0, s0).wait()
        pltpu.sync_copy(rows0, acc.at[didx.at[j]], add=True)
    plsc.subcore_barrier()
    pltpu.sync_copy(acc.at[pl.ds(sid * _STRIPE, _STRIPE)],
                    out_ref.at[pl.ds(cid * NP + sid * _STRIPE, _STRIPE)])


def _sc_spmm(xs, src_c, dst_c, zrow):
    nchunks = src_c.shape[0]
    flat = pl.kernel(
        functools.partial(_spmm_body, nchunks),
        out_type=jax.ShapeDtypeStruct((2 * NP, D), jnp.float32),
        mesh=plsc.VectorSubcoreMesh(**_MESH),
        compiler_params=pltpu.CompilerParams(use_tc_tiling_on_sc=False, needs_layout_passes=False),
        scratch_types=[
            pltpu.VMEM(((nchunks // 2 + 15) // 16, 128), jnp.int32),
            pltpu.VMEM(((nchunks // 2 + 15) // 16, 128), jnp.int32),
            pltpu.VMEM((128, D), jnp.float32),
            pltpu.VMEM((128, D), jnp.float32),
            pltpu.VMEM_SHARED((NPA, D), jnp.float32),
            pltpu.SemaphoreType.DMA,
            pltpu.SemaphoreType.DMA,
            pltpu.SemaphoreType.DMA,
        ],
    )(xs, src_c, dst_c, zrow)
    return flat.reshape(2, NP, D)


def _deg_body(src_ref, dst_ref, zn_ref, deg_ref, self_ref, srcr_ref, dstr_ref,
              se, de, srrow, drrow, deg_v, self_v, colblk, res, stage):
    cid = lax.axis_index("c")
    sid = lax.axis_index("s")
    half = EC // 2
    pre = (half + 15) // 16
    start, cnt = _chunk_range(half, sid)
    start = start + cid * half

    pltpu.sync_copy(src_ref.at[pl.ds(start, pre)], se)
    pltpu.sync_copy(dst_ref.at[pl.ds(start, pre)], de)
    pltpu.sync_copy(zn_ref, deg_v)
    pltpu.sync_copy(zn_ref, self_v)

    def chunk(j, carry):
        for g in range(8):
            s16 = se[j, pl.ds(g * 16, 16)]
            d16 = de[j, pl.ds(g * 16, 16)]
            isself = s16 == d16
            k, v, head = _seg_reduce(d16, jnp.ones((16,), jnp.float32),
                                     0.0, jnp.add)
            cur = plsc.load_gather(deg_v, [k])
            plsc.store_scatter(deg_v, [k], cur + v, mask=head)
            ks = jnp.where(isself, d16, BIG)
            k2, v2, head2 = _seg_reduce(
                ks, jnp.where(isself, 1.0, 0.0).astype(jnp.float32),
                0.0, jnp.add)
            m2 = head2 & (k2 != BIG)
            k2c = jnp.where(k2 == BIG, 0, k2)
            cur2 = plsc.load_gather(self_v, [k2c])
            plsc.store_scatter(self_v, [k2c], cur2 + v2, mask=m2)
            pos = g * 16 + _iota16()
            drrow[pl.ds(g * 16, 16)] = jnp.where(
                isself, NP + sid * 16 + _iota16(), d16)
            srrow[pl.ds(g * 16, 16)] = jnp.where(isself, j * 128 + pos, s16)
        pltpu.sync_copy(srrow, srcr_ref.at[start + j])
        pltpu.sync_copy(drrow, dstr_ref.at[start + j])
        return carry

    lax.fori_loop(0, cnt, chunk, 0)

    pltpu.sync_copy(deg_v, stage.at[sid])
    plsc.subcore_barrier()
    _combine_stripe(stage, colblk, res, deg_ref, cid, sid, jnp.add,
                    0.0)
    plsc.subcore_barrier()
    pltpu.sync_copy(self_v, stage.at[sid])
    plsc.subcore_barrier()
    _combine_stripe(stage, colblk, res, self_ref, cid, sid, jnp.add,
                    0.0)


def _sc_degself(src_c, dst_c, zn):
    pre = (EC // 2 + 15) // 16
    outs = pl.kernel(
        _deg_body,
        out_type=[jax.ShapeDtypeStruct((2 * NP,), jnp.float32),
                  jax.ShapeDtypeStruct((2 * NP,), jnp.float32),
                  jax.ShapeDtypeStruct((EC, 128), jnp.int32),
                  jax.ShapeDtypeStruct((EC, 128), jnp.int32)],
        mesh=plsc.VectorSubcoreMesh(**_MESH),
        compiler_params=pltpu.CompilerParams(use_tc_tiling_on_sc=False, needs_layout_passes=False),
        scratch_types=[
            pltpu.VMEM((pre, 128), jnp.int32),
            pltpu.VMEM((pre, 128), jnp.int32),
            pltpu.VMEM((128,), jnp.int32),
            pltpu.VMEM((128,), jnp.int32),
            pltpu.VMEM((NP,), jnp.float32),
            pltpu.VMEM((NP,), jnp.float32),
            pltpu.VMEM((16, _STRIPE), jnp.float32),
            pltpu.VMEM((_STRIPE,), jnp.float32),
            pltpu.VMEM_SHARED((16, NP), jnp.float32),
        ],
    )(src_c, dst_c, zn)
    return outs[0].reshape(2, NP), outs[1].reshape(2, NP), outs[2], outs[3]


def _scores_body(src_ref, dst_ref, a_ref, b_ref, nfill_ref, act_ref, ns_ref,
                 se, de, a_v, b_v, ns_v, actrow, colblk, res, stage):
    cid = lax.axis_index("c")
    sid = lax.axis_index("s")
    half = EC // 2
    pre = (half + 15) // 16
    start, cnt = _chunk_range(half, sid)
    start = start + cid * half

    pltpu.sync_copy(src_ref.at[pl.ds(start, pre)], se)
    pltpu.sync_copy(dst_ref.at[pl.ds(start, pre)], de)
    pltpu.sync_copy(a_ref, a_v)
    pltpu.sync_copy(b_ref, b_v)
    pltpu.sync_copy(nfill_ref, ns_v)

    def chunk(j, carry):
        for g in range(8):
            s16 = se[j, pl.ds(g * 16, 16)]
            d16 = de[j, pl.ds(g * 16, 16)]
            av = plsc.load_gather(a_v, [s16])
            bv = plsc.load_gather(b_v, [d16])
            logit = av + bv
            act16 = logit > 0.0
            sig = 1.0 / (1.0 + jnp.exp(-logit))
            sm = jnp.where(act16, sig, NEG)
            actrow[pl.ds(g * 16, 16)] = act16.astype(jnp.int32)
            for tgt in (s16, d16):
                k, v, head = _seg_reduce(tgt, sm, NEG,
                                         jnp.maximum)
                cur = plsc.load_gather(ns_v, [k])
                m = head & (v > cur)
                plsc.store_scatter(ns_v, [k], v, mask=m)
        pltpu.sync_copy(actrow, act_ref.at[start + j])
        return carry

    lax.fori_loop(0, cnt, chunk, 0)
    pltpu.sync_copy(ns_v, stage.at[sid])
    plsc.subcore_barrier()
    _combine_stripe(stage, colblk, res, ns_ref, cid, sid, jnp.maximum,
                    NEG)


def _sc_scores(src_c, dst_c, a, b, nfill):
    pre = (EC // 2 + 15) // 16
    outs = pl.kernel(
        _scores_body,
        out_type=[jax.ShapeDtypeStruct((EC, 128), jnp.int32),
                  jax.ShapeDtypeStruct((2 * NP,), jnp.float32)],
        mesh=plsc.VectorSubcoreMesh(**_MESH),
        compiler_params=pltpu.CompilerParams(use_tc_tiling_on_sc=False, needs_layout_passes=False),
        scratch_types=[
            pltpu.VMEM((pre, 128), jnp.int32),
            pltpu.VMEM((pre, 128), jnp.int32),
            pltpu.VMEM((NP,), jnp.float32),
            pltpu.VMEM((NP,), jnp.float32),
            pltpu.VMEM((NP,), jnp.float32),
            pltpu.VMEM((128,), jnp.int32),
            pltpu.VMEM((16, _STRIPE), jnp.float32),
            pltpu.VMEM((_STRIPE,), jnp.float32),
            pltpu.VMEM_SHARED((16, NP), jnp.float32),
        ],
    )(src_c, dst_c, a, b, nfill)
    return outs[0], outs[1].reshape(2, NP)


def _cc_body(src_ref, dst_ref, act_ref, lbl0_ref, maskv_ref, z8_ref,
             ones8_ref, lbl_out, vld_out,
             se, de, ae, lbl_v, mask_v, colblk, res, ones8_v, flv,
             stage, lbl_sh, flags, vld_s):
    cid = lax.axis_index("c")
    sid = lax.axis_index("s")

    @pl.when(cid == 0)
    def _():
        pre = (EC + 15) // 16
        start, cnt = _chunk_range(EC, sid)
        base = sid * _STRIPE

        pltpu.sync_copy(src_ref.at[pl.ds(start, pre)], se)
        pltpu.sync_copy(dst_ref.at[pl.ds(start, pre)], de)
        pltpu.sync_copy(act_ref.at[pl.ds(start, pre)], ae)
        pltpu.sync_copy(lbl0_ref, lbl_v)
        pltpu.sync_copy(maskv_ref.at[pl.ds(base, _STRIPE)], mask_v)
        pltpu.sync_copy(ones8_ref, ones8_v)

        def one_iter(carry):
            changed = jnp.zeros((16,), jnp.int32)

            def chunk(j, ch):
                for g in range(8):
                    s16 = se[j, pl.ds(g * 16, 16)]
                    d16 = de[j, pl.ds(g * 16, 16)]
                    a16 = ae[j, pl.ds(g * 16, 16)]
                    ls = plsc.load_gather(lbl_v, [s16])
                    ld = plsc.load_gather(lbl_v, [d16])
                    lm = jnp.where(a16 > 0, jnp.minimum(ls, ld), BIG)
                    for tgt in (s16, d16):
                        k, v, head = _seg_reduce(tgt, lm, BIG, jnp.minimum)
                        cur = plsc.load_gather(lbl_v, [k])
                        m = head & (v < cur)
                        plsc.store_scatter(lbl_v, [k], v, mask=m)
                        ch = ch | m.astype(jnp.int32)
                return ch

            changed = lax.fori_loop(0, cnt, chunk, changed)

            def halve(g, carry2):
                idx = lbl_v[pl.ds(base + g * 16, 16)]
                l2 = plsc.load_gather(lbl_v, [idx])
                lbl_v[pl.ds(base + g * 16, 16)] = l2
                return carry2

            lax.fori_loop(0, _STRIPE // 16, halve, 0)

            pltpu.sync_copy(lbl_v, stage.at[sid])
            chs = jnp.max(changed) + jnp.zeros((16,), jnp.int32)
            res[pl.ds(0, 16)] = chs
            pltpu.sync_copy(res.at[pl.ds(0, 16)], flags.at[sid])
            plsc.subcore_barrier()

            _combine_stripe(stage, colblk, res, None, cid, sid,
                            jnp.minimum, BIG)
            pltpu.sync_copy(res, lbl_sh.at[pl.ds(base, _STRIPE)])
            pltpu.sync_copy(flags, flv)
            m = flv[0, pl.ds(0, 16)]
            for r in range(1, 16):
                m = jnp.maximum(m, flv[r, pl.ds(0, 16)])
            allch = jnp.max(m)
            plsc.subcore_barrier()
            pltpu.sync_copy(lbl_sh, lbl_v)
            return allch

        lax.while_loop(lambda c: c > 0, lambda c: one_iter(c),
                       1)

        for i in range(5):
            pltpu.sync_copy(z8_ref, vld_s.at[pl.ds(base + i * 128, 128)])
        plsc.subcore_barrier()

        def vscat(g, carry):
            lbl16 = lbl_v[pl.ds(base + g * 16, 16)]
            m16 = mask_v[pl.ds(g * 16, 16)]
            tgt = jnp.where(m16 > 0, lbl16, NP + _iota16())
            pltpu.sync_copy(ones8_v, vld_s.at[tgt])
            return carry

        lax.fori_loop(0, _STRIPE // 16, vscat, 0)
        plsc.subcore_barrier()
        pltpu.sync_copy(lbl_v.at[pl.ds(base, _STRIPE)],
                        lbl_out.at[pl.ds(base, _STRIPE)])
        pltpu.sync_copy(vld_s.at[pl.ds(base, _STRIPE)],
                        vld_out.at[pl.ds(base, _STRIPE)])


def _sc_cc(src_c, dst_c, act_c, lbl0, maskv, z8, ones8):
    pre = (EC + 15) // 16
    outs = pl.kernel(
        _cc_body,
        out_type=[jax.ShapeDtypeStruct((NP,), jnp.int32),
                  jax.ShapeDtypeStruct((NP, 8), jnp.int32)],
        mesh=plsc.VectorSubcoreMesh(**_MESH),
        compiler_params=pltpu.CompilerParams(use_tc_tiling_on_sc=False, needs_layout_passes=False),
        scratch_types=[
            pltpu.VMEM((pre, 128), jnp.int32),
            pltpu.VMEM((pre, 128), jnp.int32),
            pltpu.VMEM((pre, 128), jnp.int32),
            pltpu.VMEM((NP,), jnp.int32),
            pltpu.VMEM((_STRIPE,), jnp.int32),
            pltpu.VMEM((16, _STRIPE), jnp.int32),
            pltpu.VMEM((_STRIPE,), jnp.int32),
            pltpu.VMEM((16, 8), jnp.int32),
            pltpu.VMEM((16, 16), jnp.int32),
            pltpu.VMEM_SHARED((16, NP), jnp.int32),
            pltpu.VMEM_SHARED((NP,), jnp.int32),
            pltpu.VMEM_SHARED((16, 16), jnp.int32),
            pltpu.VMEM_SHARED((NP + 16, 8), jnp.int32),
        ],
    )(src_c, dst_c, act_c, lbl0, maskv, z8, ones8)
    return outs[0], outs[1]


def _einv_body(src_ref, dst_ref, inv_ref, srcn_ref, dstn_ref,
               se, de, inv_v, rs, rd):
    cid = lax.axis_index("c")
    sid = lax.axis_index("s")
    half = EC // 2
    pre = (half + 15) // 16
    start, cnt = _chunk_range(half, sid)
    start = start + cid * half

    pltpu.sync_copy(src_ref.at[pl.ds(start, pre)], se)
    pltpu.sync_copy(dst_ref.at[pl.ds(start, pre)], de)
    pltpu.sync_copy(inv_ref, inv_v)

    def chunk(j, carry):
        for g in range(8):
            s16 = se[j, pl.ds(g * 16, 16)]
            d16 = de[j, pl.ds(g * 16, 16)]
            rs[pl.ds(g * 16, 16)] = plsc.load_gather(inv_v, [s16])
            rd[pl.ds(g * 16, 16)] = plsc.load_gather(inv_v, [d16])
        pltpu.sync_copy(rs, srcn_ref.at[start + j])
        pltpu.sync_copy(rd, dstn_ref.at[start + j])
        return carry

    lax.fori_loop(0, cnt, chunk, 0)


def _sc_einv(src_c, dst_c, inv):
    pre = (EC // 2 + 15) // 16
    return pl.kernel(
        _einv_body,
        out_type=[jax.ShapeDtypeStruct((EC, 128), jnp.int32),
                  jax.ShapeDtypeStruct((EC, 128), jnp.int32)],
        mesh=plsc.VectorSubcoreMesh(**_MESH),
        compiler_params=pltpu.CompilerParams(use_tc_tiling_on_sc=False, needs_layout_passes=False),
        scratch_types=[
            pltpu.VMEM((pre, 128), jnp.int32),
            pltpu.VMEM((pre, 128), jnp.int32),
            pltpu.VMEM((NP,), jnp.int32),
            pltpu.VMEM((128,), jnp.int32),
            pltpu.VMEM((128,), jnp.int32),
        ],
    )(src_c, dst_c, inv)


def kernel(x, edge_list, batch, W1, b1, W2, b2, W3, b3, W4, b4, W5, b5,
           pw1, pb1, pw2, pb2, fw1, fb1, fw2, fb2, y=None):
    src_c = edge_list[:, 0].astype(jnp.int32).reshape(EC, 128)
    dst_c = edge_list[:, 1].astype(jnp.int32).reshape(EC, 128)
    zrow = jnp.zeros((128, D), jnp.float32)
    zn = jnp.zeros((NP,), jnp.float32)
    nfill = jnp.full((NP,), NEG, jnp.float32)
    z8 = jnp.zeros((128, 8), jnp.int32)
    ones8 = jnp.ones((16, 8), jnp.int32)
    lbl0 = jnp.arange(NP, dtype=jnp.int32)
    onesNP = jnp.ones((NP,), jnp.int32)
    pool_src = lbl0.reshape(NP // 128, 128)
    x_p = jnp.pad(x, ((0, NP - N), (0, 0)))

    def level(h, sc, dc, Ws, pw, pb):
        deg2, self2, srcr, dstr = _sc_degself(sc, dc, zn)
        dis, iveff = _norms(deg2, self2)
        for (W, b) in Ws:
            xw, xws = _mm_conv(h, W, dis)
            acc2 = _sc_spmm(xws, srcr, dstr, zrow)
            h = _elt_post(acc2, xw, dis, iveff, b)
        pwcat = jnp.pad(jnp.concatenate([pw[:D], pw[D:]], axis=1),
                        ((0, 0), (0, 126)))
        bias2 = jnp.pad(pb.reshape(1, 1), ((0, 0), (0, 127)))
        ab = _mm(h, pwcat, bias2)
        return h, ab[:, 0], ab[:, 1]

    h, a1, b1v = level(x_p, src_c, dst_c, [(W1, b1), (W2, b2)], pw1, pb1)
    act1, ns1 = _sc_scores(src_c, dst_c, a1, b1v, nfill)
    lbl1, vld1 = _sc_cc(src_c, dst_c, act1, lbl0, onesNP, z8, ones8)
    hs = _scale_hs(h, ns1)
    accp = _sc_spmm(hs, pool_src, lbl1.reshape(NP // 128, 128), zrow)
    h2 = _add2(accp[0], accp[1])
    src2_c, dst2_c = _sc_einv(src_c, dst_c, lbl1)

    h, a2, b2v = level(h2, src2_c, dst2_c, [(W3, b3), (W4, b4)], pw2, pb2)
    act2, ns2 = _sc_scores(src2_c, dst2_c, a2, b2v, nfill)
    lbl2, vld2 = _sc_cc(src2_c, dst2_c, act2, lbl0, vld1[:, 0], z8, ones8)
    hs2 = _scale_hs(h, ns2)
    accp2 = _sc_spmm(hs2, pool_src, lbl2.reshape(NP // 128, 128), zrow)
    h3 = _add2(accp2[0], accp2[1])
    src3_c, dst3_c = _sc_einv(src2_c, dst2_c, lbl2)

    deg2, self2, srcr3, dstr3 = _sc_degself(src3_c, dst3_c, zn)
    dis3, iveff3 = _norms(deg2, self2)
    xw, xws = _mm_conv(h3, W5, dis3)
    acc2 = _sc_spmm(xws, srcr3, dstr3, zrow)
    h5 = _elt_post(acc2, xw, dis3, iveff3, b5)

    valid_col = vld2[:, 0].astype(jnp.float32).reshape(NP, 1)
    fw2p = jnp.pad(fw2, ((0, 0), (0, 127)))
    fb2p = jnp.pad(fb2.reshape(1, 1), ((0, 0), (0, 127)))
    out = _head(h5, valid_col, fw1, fb1.reshape(1, D), fw2p, fb2p)
    return out[0, 0].reshape(1)


# speedup vs baseline: 27.7433x; 1.0575x over previous
import functools

import jax
import jax.numpy as jnp
from jax import lax
from jax.experimental import pallas as pl
from jax.experimental.pallas import tpu as pltpu
from jax.experimental.pallas import tpu_sc as plsc

N = 10000
NP = 10240
NPA = NP + 256
E = 160000
EC = E // 128
D = 128
NEG = -1e30
BIG = 0x7F7F7F7F
_STRIPE = NP // 16
_ASTRIPE = NPA // 16


def _mm_body(x_ref, w_ref, b_ref, o_ref):
    o_ref[...] = jnp.dot(x_ref[...], w_ref[...],
                         preferred_element_type=jnp.float32) + b_ref[...]


def _mm(x, w, bias, bm=1024):
    k = x.shape[1]
    ncp = w.shape[1]
    return pl.pallas_call(
        _mm_body,
        grid=(NP // bm,),
        in_specs=[pl.BlockSpec((bm, k), lambda i: (i, 0)),
                  pl.BlockSpec((k, ncp), lambda i: (0, 0)),
                  pl.BlockSpec((1, ncp), lambda i: (0, 0))],
        out_specs=pl.BlockSpec((bm, ncp), lambda i: (i, 0)),
        out_shape=jax.ShapeDtypeStruct((NP, ncp), jnp.float32),
    )(x, w, bias)


def _mm_conv_body(x_ref, w_ref, dis_ref, xw_ref, xws_ref):
    t = jnp.dot(x_ref[...], w_ref[...], preferred_element_type=jnp.float32)
    xw_ref[...] = t
    xws_ref[...] = t * dis_ref[...]


def _mm_conv(x, w, dis_col, bm=1024):
    k = x.shape[1]
    return pl.pallas_call(
        _mm_conv_body,
        grid=(NP // bm,),
        in_specs=[pl.BlockSpec((bm, k), lambda i: (i, 0)),
                  pl.BlockSpec((k, D), lambda i: (0, 0)),
                  pl.BlockSpec((bm, 1), lambda i: (i, 0))],
        out_specs=[pl.BlockSpec((bm, D), lambda i: (i, 0)),
                   pl.BlockSpec((bm, D), lambda i: (i, 0))],
        out_shape=[jax.ShapeDtypeStruct((NP, D), jnp.float32),
                   jax.ShapeDtypeStruct((NP, D), jnp.float32)],
    )(x, w, dis_col)


def _elt_post_body(a_ref, b_ref, xw_ref, dis_ref, iv_ref, bias_ref, o_ref):
    agg = (a_ref[...] + b_ref[...]) * dis_ref[...]
    o_ref[...] = jnp.maximum(agg + xw_ref[...] * iv_ref[...] + bias_ref[...],
                             0.0)


def _elt_post(acc2, xw, dis_col, iveff_col, bias, bm=1024):
    return pl.pallas_call(
        _elt_post_body,
        grid=(NP // bm,),
        in_specs=[pl.BlockSpec((bm, D), lambda i: (i, 0)),
                  pl.BlockSpec((bm, D), lambda i: (i, 0)),
                  pl.BlockSpec((bm, D), lambda i: (i, 0)),
                  pl.BlockSpec((bm, 1), lambda i: (i, 0)),
                  pl.BlockSpec((bm, 1), lambda i: (i, 0)),
                  pl.BlockSpec((1, D), lambda i: (0, 0))],
        out_specs=pl.BlockSpec((bm, D), lambda i: (i, 0)),
        out_shape=jax.ShapeDtypeStruct((NP, D), jnp.float32),
    )(acc2[0], acc2[1], xw, dis_col, iveff_col, bias.reshape(1, D))


def _add2_body(a_ref, b_ref, o_ref):
    o_ref[...] = a_ref[...] + b_ref[...]


def _add2(a, b, bm=1024):
    return pl.pallas_call(
        _add2_body,
        grid=(NP // bm,),
        in_specs=[pl.BlockSpec((bm, D), lambda i: (i, 0)),
                  pl.BlockSpec((bm, D), lambda i: (i, 0))],
        out_specs=pl.BlockSpec((bm, D), lambda i: (i, 0)),
        out_shape=jax.ShapeDtypeStruct((NP, D), jnp.float32),
    )(a, b)


def _norms_body(da_ref, db_ref, sa_ref, sb_ref, dis_ref, iv_ref):
    deg = da_ref[...] + db_ref[...] + 1.0
    selfc = sa_ref[...] + sb_ref[...]
    dis = lax.rsqrt(deg)
    dis_ref[...] = dis
    iv_ref[...] = (dis * dis) * (1.0 + selfc)


def _norms(deg2, self2):
    outs = pl.pallas_call(
        _norms_body,
        in_specs=[pl.BlockSpec((80, 128), lambda: (0, 0))] * 4,
        out_specs=[pl.BlockSpec((80, 128), lambda: (0, 0))] * 2,
        out_shape=[jax.ShapeDtypeStruct((80, 128), jnp.float32)] * 2,
    )(deg2[0].reshape(80, 128), deg2[1].reshape(80, 128),
      self2[0].reshape(80, 128), self2[1].reshape(80, 128))
    return outs[0].reshape(NP, 1), outs[1].reshape(NP, 1)


def _scale_body(h_ref, na_ref, nb_ref, o_ref):
    ns = jnp.maximum(na_ref[...], nb_ref[...])
    ns = jnp.where(ns <= NEG * 0.5, 1.0, ns)
    o_ref[...] = h_ref[...] * ns


def _scale_hs(h, ns2, bm=1024):
    return pl.pallas_call(
        _scale_body,
        grid=(NP // bm,),
        in_specs=[pl.BlockSpec((bm, D), lambda i: (i, 0)),
                  pl.BlockSpec((bm, 1), lambda i: (i, 0)),
                  pl.BlockSpec((bm, 1), lambda i: (i, 0))],
        out_specs=pl.BlockSpec((bm, D), lambda i: (i, 0)),
        out_shape=jax.ShapeDtypeStruct((NP, D), jnp.float32),
    )(h, ns2[0].reshape(NP, 1), ns2[1].reshape(NP, 1))


def _head_body(h_ref, v_ref, fw1_ref, fb1_ref, fw2_ref, fb2_ref, o_ref,
               acc_ref):
    p = pl.program_id(0)
    bm = h_ref.shape[0]

    @pl.when(p == 0)
    def _():
        acc_ref[...] = jnp.zeros_like(acc_ref)

    rows = lax.broadcasted_iota(jnp.int32, (bm, 1), 0) + p * bm
    vf = jnp.where(rows < N, v_ref[...], 0.0)
    hv = h_ref[...] * vf
    acc_ref[0:1, :] += jnp.sum(hv, axis=0, keepdims=True)
    acc_ref[1:2, :] += jnp.sum(vf)

    @pl.when(p == pl.num_programs(0) - 1)
    def _():
        cnt = acc_ref[1, 0]
        g = acc_ref[0:1, :] / jnp.maximum(cnt, 1.0)
        z = jnp.maximum(
            jnp.dot(g, fw1_ref[...], preferred_element_type=jnp.float32)
            + fb1_ref[...], 0.0)
        o = jnp.dot(z, fw2_ref[...], preferred_element_type=jnp.float32)
        o = 1.0 / (1.0 + jnp.exp(-(o + fb2_ref[...])))
        o_ref[...] = o


def _head(h, valid_col, fw1, fb1, fw2p, fb2p, bm=1024):
    return pl.pallas_call(
        _head_body,
        grid=(NP // bm,),
        in_specs=[pl.BlockSpec((bm, D), lambda i: (i, 0)),
                  pl.BlockSpec((bm, 1), lambda i: (i, 0)),
                  pl.BlockSpec((D, D), lambda i: (0, 0)),
                  pl.BlockSpec((1, D), lambda i: (0, 0)),
                  pl.BlockSpec((D, D), lambda i: (0, 0)),
                  pl.BlockSpec((1, D), lambda i: (0, 0))],
        out_specs=pl.BlockSpec((1, D), lambda i: (0, 0)),
        out_shape=jax.ShapeDtypeStruct((1, D), jnp.float32),
        scratch_shapes=[pltpu.VMEM((8, D), jnp.float32)],
    )(h, valid_col, fw1, fb1, fw2p, fb2p)


_MESH = dict(core_axis_name="c", subcore_axis_name="s")


def _iota16():
    return lax.iota(jnp.int32, 16)


def _shift(x, sh, fill):
    it = _iota16()
    idx = jnp.minimum(it + sh, 15)
    g = x.at[idx].get(mode="promise_in_bounds")
    return jnp.where(it < 16 - sh, g, fill)


def _seg_reduce(keys, vals, neutral, op):
    k, v = plsc.sort_key_val(keys, vals)
    for sh in (1, 2, 4, 8):
        k_sh = _shift(k, sh, jnp.int32(-1))
        v_sh = _shift(v, sh, neutral)
        v = jnp.where(k_sh == k, op(v, v_sh), v)
    it = _iota16()
    kp = k.at[jnp.maximum(it - 1, 0)].get(mode="promise_in_bounds")
    head = (it == 0) | (kp != k)
    return k, v, head


def _chunk_range(half, sid):
    start = (sid * half) // 16
    cnt = ((sid + 1) * half) // 16 - start
    return start, cnt


def _combine_stripe(stage, colblk, res, out_ref, cid, sid, op, neutral):
    base = sid * _STRIPE
    pltpu.sync_copy(stage.at[:, pl.ds(base, _STRIPE)], colblk)

    def body(g, carry):
        a = colblk[0, pl.ds(g * 16, 16)]
        for r in range(1, 16):
            a = op(a, colblk[r, pl.ds(g * 16, 16)])
        res[pl.ds(g * 16, 16)] = a
        return carry

    lax.fori_loop(0, _STRIPE // 16, body, 0)
    if out_ref is not None:
        pltpu.sync_copy(res, out_ref.at[pl.ds(cid * NP + base, _STRIPE)])


def _spmm_body(nchunks, ordered, xs_ref, src_ref, dst_ref, zrow_ref, out_ref,
               sidx, didx, rows0, rows1, dredir, acc, s0, s1, semz):
    cid = lax.axis_index("c")
    sid = lax.axis_index("s")
    half = nchunks // 2
    pre = nchunks if ordered else (half + 15) // 16
    nz = 336 if ordered else _ASTRIPE
    astart = sid * nz

    for i in range(nz // 128):
        pltpu.async_copy(zrow_ref, acc.at[pl.ds(astart + i * 128, 128)],
                         semz).wait()
    if nz % 128:
        pltpu.async_copy(zrow_ref.at[pl.ds(0, nz % 128)],
                         acc.at[pl.ds(astart + 128 * (nz // 128), nz % 128)],
                         semz).wait()

    if ordered:
        halfn = NP // 2
        lo = (cid * 16 + sid) * 320
        pltpu.sync_copy(dst_ref.at[pl.ds(0, pre)], didx)
        plsc.subcore_barrier()

        def obody(j, carry):
            anyown = jnp.int32(0)
            for g in range(8):
                d = didx[j, pl.ds(g * 16, 16)]
                own = (d >= lo) & (d < lo + 320)
                dredir[pl.ds(g * 16, 16)] = jnp.where(
                    own, d - cid * halfn, halfn + sid * 16 + _iota16())
                anyown = anyown | jnp.max(own.astype(jnp.int32))

            @pl.when(anyown > 0)
            def _():
                pltpu.sync_copy(xs_ref.at[pl.ds(j * 128, 128)], rows0)
                pltpu.sync_copy(rows0, acc.at[dredir], add=True)
            return carry

        lax.fori_loop(0, nchunks, obody, 0)
        plsc.subcore_barrier()
        pltpu.sync_copy(
            acc.at[pl.ds(sid * 320, 320)],
            out_ref.at[pl.ds(cid * NP + cid * halfn + sid * 320, 320)])
        return

    start, cnt = _chunk_range(half, sid)
    start = start + cid * half
    pltpu.sync_copy(src_ref.at[pl.ds(start, pre)], sidx)
    pltpu.sync_copy(dst_ref.at[pl.ds(start, pre)], didx)
    plsc.subcore_barrier()

    def mindst(j):
        m = didx[j, pl.ds(0, 16)]
        for g in range(1, 8):
            m = jnp.minimum(m, didx[j, pl.ds(g * 16, 16)])
        return jnp.min(m)

    def body(p, carry):
        j = 2 * p
        both = jnp.minimum(mindst(j), mindst(j + 1)) < NP

        @pl.when(both)
        def _():
            h0 = pltpu.async_copy(xs_ref.at[sidx.at[j]], rows0, s0)
            h1 = pltpu.async_copy(xs_ref.at[sidx.at[j + 1]], rows1, s1)
            h0.wait()
            pltpu.sync_copy(rows0, acc.at[didx.at[j]], add=True)
            h1.wait()
            pltpu.sync_copy(rows1, acc.at[didx.at[j + 1]], add=True)
        return carry

    lax.fori_loop(0, cnt // 2, body, 0)

    @pl.when((cnt % 2 == 1) & (mindst(cnt - 1) < NP))
    def _():
        j = cnt - 1
        pltpu.async_copy(xs_ref.at[sidx.at[j]], rows0, s0).wait()
        pltpu.sync_copy(rows0, acc.at[didx.at[j]], add=True)
    plsc.subcore_barrier()
    pltpu.sync_copy(acc.at[pl.ds(sid * _STRIPE, _STRIPE)],
                    out_ref.at[pl.ds(cid * NP + sid * _STRIPE, _STRIPE)])


def _sc_spmm(xs, src_c, dst_c, zrow, ordered=False):
    nchunks = src_c.shape[0]
    flat = pl.kernel(
        functools.partial(_spmm_body, nchunks, ordered),
        out_type=jax.ShapeDtypeStruct((2 * NP, D), jnp.float32),
        mesh=plsc.VectorSubcoreMesh(**_MESH),
        compiler_params=pltpu.CompilerParams(use_tc_tiling_on_sc=False, needs_layout_passes=False),
        scratch_types=[
            pltpu.VMEM((1 if ordered else (nchunks // 2 + 15) // 16, 128), jnp.int32),
            pltpu.VMEM((nchunks if ordered else (nchunks // 2 + 15) // 16, 128), jnp.int32),
            pltpu.VMEM((128, D), jnp.float32),
            pltpu.VMEM((128, D), jnp.float32),
            pltpu.VMEM((128,), jnp.int32),
            pltpu.VMEM_SHARED((5376 if ordered else NPA, D), jnp.float32),
            pltpu.SemaphoreType.DMA,
            pltpu.SemaphoreType.DMA,
            pltpu.SemaphoreType.DMA,
        ],
    )(xs, src_c, dst_c, zrow)
    return flat.reshape(2, NP, D)


def _deg_body(src_ref, dst_ref, zn_ref, deg_ref, self_ref, srcr_ref, dstr_ref,
              se, de, srrow, drrow, deg_v, self_v, colblk, res, stage):
    cid = lax.axis_index("c")
    sid = lax.axis_index("s")
    half = EC // 2
    pre = (half + 15) // 16
    start, cnt = _chunk_range(half, sid)
    start = start + cid * half

    pltpu.sync_copy(src_ref.at[pl.ds(start, pre)], se)
    pltpu.sync_copy(dst_ref.at[pl.ds(start, pre)], de)
    pltpu.sync_copy(zn_ref, deg_v)
    pltpu.sync_copy(zn_ref, self_v)

    def chunk(j, carry):
        for g in range(8):
            s16 = se[j, pl.ds(g * 16, 16)]
            d16 = de[j, pl.ds(g * 16, 16)]
            isself = s16 == d16
            k, v, head = _seg_reduce(d16, jnp.ones((16,), jnp.float32),
                                     0.0, jnp.add)
            cur = plsc.load_gather(deg_v, [k])
            plsc.store_scatter(deg_v, [k], cur + v, mask=head)
            ks = jnp.where(isself, d16, BIG)
            k2, v2, head2 = _seg_reduce(
                ks, jnp.where(isself, 1.0, 0.0).astype(jnp.float32),
                0.0, jnp.add)
            m2 = head2 & (k2 != BIG)
            k2c = jnp.where(k2 == BIG, 0, k2)
            cur2 = plsc.load_gather(self_v, [k2c])
            plsc.store_scatter(self_v, [k2c], cur2 + v2, mask=m2)
            pos = g * 16 + _iota16()
            drrow[pl.ds(g * 16, 16)] = jnp.where(
                isself, NP + sid * 16 + _iota16(), d16)
            srrow[pl.ds(g * 16, 16)] = jnp.where(isself, j * 128 + pos, s16)
        pltpu.sync_copy(srrow, srcr_ref.at[start + j])
        pltpu.sync_copy(drrow, dstr_ref.at[start + j])
        return carry

    lax.fori_loop(0, cnt, chunk, 0)

    pltpu.sync_copy(deg_v, stage.at[sid])
    plsc.subcore_barrier()
    _combine_stripe(stage, colblk, res, deg_ref, cid, sid, jnp.add,
                    0.0)
    plsc.subcore_barrier()
    pltpu.sync_copy(self_v, stage.at[sid])
    plsc.subcore_barrier()
    _combine_stripe(stage, colblk, res, self_ref, cid, sid, jnp.add,
                    0.0)


def _sc_degself(src_c, dst_c, zn):
    pre = (EC // 2 + 15) // 16
    outs = pl.kernel(
        _deg_body,
        out_type=[jax.ShapeDtypeStruct((2 * NP,), jnp.float32),
                  jax.ShapeDtypeStruct((2 * NP,), jnp.float32),
                  jax.ShapeDtypeStruct((EC, 128), jnp.int32),
                  jax.ShapeDtypeStruct((EC, 128), jnp.int32)],
        mesh=plsc.VectorSubcoreMesh(**_MESH),
        compiler_params=pltpu.CompilerParams(use_tc_tiling_on_sc=False, needs_layout_passes=False),
        scratch_types=[
            pltpu.VMEM((pre, 128), jnp.int32),
            pltpu.VMEM((pre, 128), jnp.int32),
            pltpu.VMEM((128,), jnp.int32),
            pltpu.VMEM((128,), jnp.int32),
            pltpu.VMEM((NP,), jnp.float32),
            pltpu.VMEM((NP,), jnp.float32),
            pltpu.VMEM((16, _STRIPE), jnp.float32),
            pltpu.VMEM((_STRIPE,), jnp.float32),
            pltpu.VMEM_SHARED((16, NP), jnp.float32),
        ],
    )(src_c, dst_c, zn)
    return outs[0].reshape(2, NP), outs[1].reshape(2, NP), outs[2], outs[3]


def _scores_body(src_ref, dst_ref, a_ref, b_ref, nfill_ref, act_ref, ns_ref,
                 se, de, a_v, b_v, ns_v, actrow, colblk, res, stage):
    cid = lax.axis_index("c")
    sid = lax.axis_index("s")
    half = EC // 2
    pre = (half + 15) // 16
    start, cnt = _chunk_range(half, sid)
    start = start + cid * half

    pltpu.sync_copy(src_ref.at[pl.ds(start, pre)], se)
    pltpu.sync_copy(dst_ref.at[pl.ds(start, pre)], de)
    pltpu.sync_copy(a_ref, a_v)
    pltpu.sync_copy(b_ref, b_v)
    pltpu.sync_copy(nfill_ref, ns_v)

    def chunk(j, carry):
        for g in range(8):
            s16 = se[j, pl.ds(g * 16, 16)]
            d16 = de[j, pl.ds(g * 16, 16)]
            av = plsc.load_gather(a_v, [s16])
            bv = plsc.load_gather(b_v, [d16])
            logit = av + bv
            act16 = logit > 0.0
            sig = 1.0 / (1.0 + jnp.exp(-logit))
            sm = jnp.where(act16, sig, NEG)
            actrow[pl.ds(g * 16, 16)] = act16.astype(jnp.int32)
            for tgt in (s16, d16):
                k, v, head = _seg_reduce(tgt, sm, NEG,
                                         jnp.maximum)
                cur = plsc.load_gather(ns_v, [k])
                m = head & (v > cur)
                plsc.store_scatter(ns_v, [k], v, mask=m)
        pltpu.sync_copy(actrow, act_ref.at[start + j])
        return carry

    lax.fori_loop(0, cnt, chunk, 0)
    pltpu.sync_copy(ns_v, stage.at[sid])
    plsc.subcore_barrier()
    _combine_stripe(stage, colblk, res, ns_ref, cid, sid, jnp.maximum,
                    NEG)


def _sc_scores(src_c, dst_c, a, b, nfill):
    pre = (EC // 2 + 15) // 16
    outs = pl.kernel(
        _scores_body,
        out_type=[jax.ShapeDtypeStruct((EC, 128), jnp.int32),
                  jax.ShapeDtypeStruct((2 * NP,), jnp.float32)],
        mesh=plsc.VectorSubcoreMesh(**_MESH),
        compiler_params=pltpu.CompilerParams(use_tc_tiling_on_sc=False, needs_layout_passes=False),
        scratch_types=[
            pltpu.VMEM((pre, 128), jnp.int32),
            pltpu.VMEM((pre, 128), jnp.int32),
            pltpu.VMEM((NP,), jnp.float32),
            pltpu.VMEM((NP,), jnp.float32),
            pltpu.VMEM((NP,), jnp.float32),
            pltpu.VMEM((128,), jnp.int32),
            pltpu.VMEM((16, _STRIPE), jnp.float32),
            pltpu.VMEM((_STRIPE,), jnp.float32),
            pltpu.VMEM_SHARED((16, NP), jnp.float32),
        ],
    )(src_c, dst_c, a, b, nfill)
    return outs[0], outs[1].reshape(2, NP)


def _cc_body(src_ref, dst_ref, act_ref, lbl0_ref, maskv_ref, z8_ref,
             ones8_ref, lbl_out, vld_out,
             se, de, ae, lbl_v, mask_v, colblk, res, ones8_v, flv,
             stage, lbl_sh, flags, vld_s):
    cid = lax.axis_index("c")
    sid = lax.axis_index("s")

    @pl.when(cid == 0)
    def _():
        pre = (EC + 15) // 16
        start, cnt = _chunk_range(EC, sid)
        base = sid * _STRIPE

        pltpu.sync_copy(src_ref.at[pl.ds(start, pre)], se)
        pltpu.sync_copy(dst_ref.at[pl.ds(start, pre)], de)
        pltpu.sync_copy(act_ref.at[pl.ds(start, pre)], ae)
        pltpu.sync_copy(lbl0_ref, lbl_v)
        pltpu.sync_copy(maskv_ref.at[pl.ds(base, _STRIPE)], mask_v)
        pltpu.sync_copy(ones8_ref, ones8_v)

        def one_iter(carry):
            changed = jnp.zeros((16,), jnp.int32)

            def chunk(j, ch):
                for g in range(8):
                    s16 = se[j, pl.ds(g * 16, 16)]
                    d16 = de[j, pl.ds(g * 16, 16)]
                    a16 = ae[j, pl.ds(g * 16, 16)]
                    ls = plsc.load_gather(lbl_v, [s16])
                    ld = plsc.load_gather(lbl_v, [d16])
                    lm = jnp.where(a16 > 0, jnp.minimum(ls, ld), BIG)
                    for tgt in (s16, d16):
                        k, v, head = _seg_reduce(tgt, lm, BIG, jnp.minimum)
                        cur = plsc.load_gather(lbl_v, [k])
                        m = head & (v < cur)
                        plsc.store_scatter(lbl_v, [k], v, mask=m)
                        ch = ch | m.astype(jnp.int32)
                return ch

            changed = lax.fori_loop(0, cnt, chunk, changed)

            def halve(g, carry2):
                idx = lbl_v[pl.ds(base + g * 16, 16)]
                l2 = plsc.load_gather(lbl_v, [idx])
                lbl_v[pl.ds(base + g * 16, 16)] = l2
                return carry2

            lax.fori_loop(0, _STRIPE // 16, halve, 0)

            pltpu.sync_copy(lbl_v, stage.at[sid])
            chs = jnp.max(changed) + jnp.zeros((16,), jnp.int32)
            res[pl.ds(0, 16)] = chs
            pltpu.sync_copy(res.at[pl.ds(0, 16)], flags.at[sid])
            plsc.subcore_barrier()

            _combine_stripe(stage, colblk, res, None, cid, sid,
                            jnp.minimum, BIG)
            pltpu.sync_copy(res, lbl_sh.at[pl.ds(base, _STRIPE)])
            pltpu.sync_copy(flags, flv)
            m = flv[0, pl.ds(0, 16)]
            for r in range(1, 16):
                m = jnp.maximum(m, flv[r, pl.ds(0, 16)])
            allch = jnp.max(m)
            plsc.subcore_barrier()
            pltpu.sync_copy(lbl_sh, lbl_v)
            return allch

        lax.while_loop(lambda c: c > 0, lambda c: one_iter(c),
                       1)

        for i in range(5):
            pltpu.sync_copy(z8_ref, vld_s.at[pl.ds(base + i * 128, 128)])
        plsc.subcore_barrier()

        def vscat(g, carry):
            lbl16 = lbl_v[pl.ds(base + g * 16, 16)]
            m16 = mask_v[pl.ds(g * 16, 16)]
            tgt = jnp.where(m16 > 0, lbl16, NP + _iota16())
            pltpu.sync_copy(ones8_v, vld_s.at[tgt])
            return carry

        lax.fori_loop(0, _STRIPE // 16, vscat, 0)
        plsc.subcore_barrier()
        pltpu.sync_copy(lbl_v.at[pl.ds(base, _STRIPE)],
                        lbl_out.at[pl.ds(base, _STRIPE)])
        pltpu.sync_copy(vld_s.at[pl.ds(base, _STRIPE)],
                        vld_out.at[pl.ds(base, _STRIPE)])


def _sc_cc(src_c, dst_c, act_c, lbl0, maskv, z8, ones8):
    pre = (EC + 15) // 16
    outs = pl.kernel(
        _cc_body,
        out_type=[jax.ShapeDtypeStruct((NP,), jnp.int32),
                  jax.ShapeDtypeStruct((NP, 8), jnp.int32)],
        mesh=plsc.VectorSubcoreMesh(**_MESH),
        compiler_params=pltpu.CompilerParams(use_tc_tiling_on_sc=False, needs_layout_passes=False),
        scratch_types=[
            pltpu.VMEM((pre, 128), jnp.int32),
            pltpu.VMEM((pre, 128), jnp.int32),
            pltpu.VMEM((pre, 128), jnp.int32),
            pltpu.VMEM((NP,), jnp.int32),
            pltpu.VMEM((_STRIPE,), jnp.int32),
            pltpu.VMEM((16, _STRIPE), jnp.int32),
            pltpu.VMEM((_STRIPE,), jnp.int32),
            pltpu.VMEM((16, 8), jnp.int32),
            pltpu.VMEM((16, 16), jnp.int32),
            pltpu.VMEM_SHARED((16, NP), jnp.int32),
            pltpu.VMEM_SHARED((NP,), jnp.int32),
            pltpu.VMEM_SHARED((16, 16), jnp.int32),
            pltpu.VMEM_SHARED((NP + 16, 8), jnp.int32),
        ],
    )(src_c, dst_c, act_c, lbl0, maskv, z8, ones8)
    return outs[0], outs[1]


def _einv_body(src_ref, dst_ref, inv_ref, srcn_ref, dstn_ref,
               se, de, inv_v, rs, rd):
    cid = lax.axis_index("c")
    sid = lax.axis_index("s")
    half = EC // 2
    pre = (half + 15) // 16
    start, cnt = _chunk_range(half, sid)
    start = start + cid * half

    pltpu.sync_copy(src_ref.at[pl.ds(start, pre)], se)
    pltpu.sync_copy(dst_ref.at[pl.ds(start, pre)], de)
    pltpu.sync_copy(inv_ref, inv_v)

    def chunk(j, carry):
        for g in range(8):
            s16 = se[j, pl.ds(g * 16, 16)]
            d16 = de[j, pl.ds(g * 16, 16)]
            rs[pl.ds(g * 16, 16)] = plsc.load_gather(inv_v, [s16])
            rd[pl.ds(g * 16, 16)] = plsc.load_gather(inv_v, [d16])
        pltpu.sync_copy(rs, srcn_ref.at[start + j])
        pltpu.sync_copy(rd, dstn_ref.at[start + j])
        return carry

    lax.fori_loop(0, cnt, chunk, 0)


def _sc_einv(src_c, dst_c, inv):
    pre = (EC // 2 + 15) // 16
    return pl.kernel(
        _einv_body,
        out_type=[jax.ShapeDtypeStruct((EC, 128), jnp.int32),
                  jax.ShapeDtypeStruct((EC, 128), jnp.int32)],
        mesh=plsc.VectorSubcoreMesh(**_MESH),
        compiler_params=pltpu.CompilerParams(use_tc_tiling_on_sc=False, needs_layout_passes=False),
        scratch_types=[
            pltpu.VMEM((pre, 128), jnp.int32),
            pltpu.VMEM((pre, 128), jnp.int32),
            pltpu.VMEM((NP,), jnp.int32),
            pltpu.VMEM((128,), jnp.int32),
            pltpu.VMEM((128,), jnp.int32),
        ],
    )(src_c, dst_c, inv)


def kernel(x, edge_list, batch, W1, b1, W2, b2, W3, b3, W4, b4, W5, b5,
           pw1, pb1, pw2, pb2, fw1, fb1, fw2, fb2, y=None):
    src_c = edge_list[:, 0].astype(jnp.int32).reshape(EC, 128)
    dst_c = edge_list[:, 1].astype(jnp.int32).reshape(EC, 128)
    zrow = jnp.zeros((128, D), jnp.float32)
    zn = jnp.zeros((NP,), jnp.float32)
    nfill = jnp.full((NP,), NEG, jnp.float32)
    z8 = jnp.zeros((128, 8), jnp.int32)
    ones8 = jnp.ones((16, 8), jnp.int32)
    lbl0 = jnp.arange(NP, dtype=jnp.int32)
    onesNP = jnp.ones((NP,), jnp.int32)
    pool_src = lbl0.reshape(NP // 128, 128)
    x_p = jnp.pad(x, ((0, NP - N), (0, 0)))

    def level(h, sc, dc, Ws, pw, pb):
        deg2, self2, srcr, dstr = _sc_degself(sc, dc, zn)
        dis, iveff = _norms(deg2, self2)
        for (W, b) in Ws:
            xw, xws = _mm_conv(h, W, dis)
            acc2 = _sc_spmm(xws, srcr, dstr, zrow)
            h = _elt_post(acc2, xw, dis, iveff, b)
        pwcat = jnp.pad(jnp.concatenate([pw[:D], pw[D:]], axis=1),
                        ((0, 0), (0, 126)))
        bias2 = jnp.pad(pb.reshape(1, 1), ((0, 0), (0, 127)))
        ab = _mm(h, pwcat, bias2)
        return h, ab[:, 0], ab[:, 1]

    h, a1, b1v = level(x_p, src_c, dst_c, [(W1, b1), (W2, b2)], pw1, pb1)
    act1, ns1 = _sc_scores(src_c, dst_c, a1, b1v, nfill)
    lbl1, vld1 = _sc_cc(src_c, dst_c, act1, lbl0, onesNP, z8, ones8)
    hs = _scale_hs(h, ns1)
    accp = _sc_spmm(hs, pool_src, lbl1.reshape(NP // 128, 128), zrow)
    h2 = _add2(accp[0], accp[1])
    src2_c, dst2_c = _sc_einv(src_c, dst_c, lbl1)

    h, a2, b2v = level(h2, src2_c, dst2_c, [(W3, b3), (W4, b4)], pw2, pb2)
    act2, ns2 = _sc_scores(src2_c, dst2_c, a2, b2v, nfill)
    lbl2, vld2 = _sc_cc(src2_c, dst2_c, act2, lbl0, vld1[:, 0], z8, ones8)
    hs2 = _scale_hs(h, ns2)
    accp2 = _sc_spmm(hs2, pool_src, lbl2.reshape(NP // 128, 128), zrow)
    h3 = _add2(accp2[0], accp2[1])
    src3_c, dst3_c = _sc_einv(src2_c, dst2_c, lbl2)

    deg2, self2, srcr3, dstr3 = _sc_degself(src3_c, dst3_c, zn)
    dis3, iveff3 = _norms(deg2, self2)
    xw, xws = _mm_conv(h3, W5, dis3)
    acc2 = _sc_spmm(xws, srcr3, dstr3, zrow)
    h5 = _elt_post(acc2, xw, dis3, iveff3, b5)

    valid_col = vld2[:, 0].astype(jnp.float32).reshape(NP, 1)
    fw2p = jnp.pad(fw2, ((0, 0), (0, 127)))
    fb2p = jnp.pad(fb2.reshape(1, 1), ((0, 0), (0, 127)))
    out = _head(h5, valid_col, fw1, fb1.reshape(1, D), fw2p, fb2p)
    return out[0, 0].reshape(1)
